# Initial kernel scaffold; baseline (speedup 1.0000x reference)
import jax
import jax.numpy as jnp
from jax.experimental import pallas as pl


def kernel(x, edge_index, batch, W_g1, b_g1, W_g2, b_g2, W_proj, b_proj, W_p1, b_p1, W_p2, b_p2, W_m1, b_m1, W_m2, b_m2):
    raise NotImplementedError("write your pallas kernel here")


# speedup vs baseline: 19.3645x; 19.3645x over previous
import jax
import jax.numpy as jnp
from jax import lax
from jax.experimental import pallas as pl
from jax.experimental.pallas import tpu as pltpu
from jax.experimental.pallas import tpu_sc as plsc

F32 = jnp.float32
I32 = jnp.int32

NC, NS, L = 2, 16, 16
NW = NC * NS

NB = 16
KC = 100
CC = NB * KC
KM_IT = 5

EW = 80


def _mesh():
    return plsc.VectorSubcoreMesh(core_axis_name="c", subcore_axis_name="s")


def _worker_id():
    return lax.axis_index("s") * NC + lax.axis_index("c")


def _sc_deg(n_nodes, n_edges):
    ept = n_edges // NW
    rpt = ept // EW

    def body(dst_hbm, out_hbm, stage, hist):
        w = _worker_id()
        zero = jnp.zeros((L,), F32)

        def zr(i, _):
            hist[pl.ds(i * L, L)] = zero
            return 0

        lax.fori_loop(0, n_nodes // L, zr, 0)
        pltpu.sync_copy(dst_hbm.at[w], stage)
        ones = jnp.ones((L,), F32)

        def row(r, _):
            for g in range(EW // L):
                idx = stage[r, pl.ds(g * L, L)]
                plsc.addupdate_scatter(hist, [idx], ones)
            return 0

        lax.fori_loop(0, rpt, row, 0)
        pltpu.sync_copy(hist, out_hbm.at[pl.ds(w * n_nodes, n_nodes)])

    return pl.kernel(
        body,
        out_type=jax.ShapeDtypeStruct((NW * n_nodes,), F32),
        mesh=_mesh(),
        compiler_params=pltpu.CompilerParams(needs_layout_passes=False),
        scratch_types=[
            pltpu.VMEM((rpt, EW), I32),
            pltpu.VMEM((n_nodes,), F32),
        ],
    )


def _sc_msg(n_nodes, n_edges, d):
    CH = 50
    SB = 8
    ept = n_edges // NW
    rpt = ept // CH
    nsb = rpt // SB
    ZB = 8
    nblk = n_nodes // ZB
    kmax = (nblk + NS - 1) // NS

    def body(y_hbm, src_hbm, dst_hbm, out_hbm,
             sidx, didx, rows0, rows1, zbuf, acc, sem0, sem1):
        cid = lax.axis_index("c")
        sid = lax.axis_index("s")
        w = sid * NC + cid
        zero = jnp.zeros((L,), F32)
        dl = d // L

        def zb(i, _):
            zbuf[i // dl, pl.ds((i % dl) * L, L)] = zero
            return 0

        lax.fori_loop(0, ZB * dl, zb, 0)

        def zblk(k, _):
            b = k * NS + sid

            @pl.when(b < nblk)
            def _():
                pltpu.sync_copy(zbuf, acc.at[pl.ds(b * ZB, ZB)])
            return 0

        lax.fori_loop(0, kmax, zblk, 0)
        plsc.subcore_barrier()

        bufs = (rows0, rows1)
        sems = (sem0, sem1)

        def sblk(k, _):
            pltpu.sync_copy(src_hbm.at[w, pl.ds(k * SB, SB)], sidx)
            pltpu.sync_copy(dst_hbm.at[w, pl.ds(k * SB, SB)], didx)
            pltpu.async_copy(y_hbm.at[sidx.at[0]], bufs[0], sems[0])
            for j in range(SB):
                b = j % 2
                pltpu.make_async_copy(y_hbm.at[sidx.at[j]], bufs[b], sems[b]).wait()
                if j + 1 < SB:
                    pltpu.async_copy(y_hbm.at[sidx.at[j + 1]], bufs[1 - b],
                                     sems[1 - b])
                pltpu.sync_copy(bufs[b], acc.at[didx.at[j]], add=True)
            return 0

        lax.fori_loop(0, nsb, sblk, 0)
        plsc.subcore_barrier()

        def dblk(k, _):
            b = k * NS + sid

            @pl.when(b < nblk)
            def _():
                pltpu.sync_copy(acc.at[pl.ds(b * ZB, ZB)],
                                out_hbm.at[cid, pl.ds(b * ZB, ZB)])
            return 0

        lax.fori_loop(0, kmax, dblk, 0)

    return pl.kernel(
        body,
        out_type=jax.ShapeDtypeStruct((NC, n_nodes, d), F32),
        mesh=_mesh(),
        compiler_params=pltpu.CompilerParams(needs_layout_passes=False),
        scratch_types=[
            pltpu.VMEM((SB, CH), I32),
            pltpu.VMEM((SB, CH), I32),
            pltpu.VMEM((CH, d), F32),
            pltpu.VMEM((CH, d), F32),
            pltpu.VMEM((ZB, d), F32),
            pltpu.VMEM_SHARED((n_nodes, d), F32),
            pltpu.SemaphoreType.DMA,
            pltpu.SemaphoreType.DMA,
        ],
    )


def _sc_stats(n_nodes, d, n_clusters, n_edges):
    nrow = n_nodes // EW
    ept = n_edges // NW
    erpt = ept // EW
    ZB = 80
    nblk = n_clusters // ZB
    kmax = (nblk + NS - 1) // NS

    def body(h_hbm, clu_hbm, srcg, dstg,
             out_hbm, cnt_hbm, eid_hbm,
             cstage, rowbuf, hist, clu_v, sstage, dstage, zbuf, acc):
        cid = lax.axis_index("c")
        sid = lax.axis_index("s")
        w = sid * NC + cid
        zero = jnp.zeros((L,), F32)
        dl = d // L

        def zb(i, _):
            zbuf[i // dl, pl.ds((i % dl) * L, L)] = zero
            return 0

        lax.fori_loop(0, ZB * dl, zb, 0)

        def zblk(k, _):
            b = k * NS + sid

            @pl.when(b < nblk)
            def _():
                pltpu.sync_copy(zbuf, acc.at[pl.ds(b * ZB, ZB)])
            return 0

        lax.fori_loop(0, kmax, zblk, 0)

        def zh(i, _):
            hist[pl.ds(i * L, L)] = zero
            return 0

        lax.fori_loop(0, n_clusters // L, zh, 0)
        pltpu.sync_copy(clu_hbm, clu_v)
        plsc.subcore_barrier()

        ones = jnp.ones((L,), F32)
        lo = (w * nrow) // NW
        hi = ((w + 1) * nrow) // NW

        def row(r, _):
            for g in range(EW // L):
                v = clu_v[pl.ds(r * EW + g * L, L)]
                cstage[pl.ds(g * L, L)] = v
                plsc.addupdate_scatter(hist, [v], ones)
            pltpu.sync_copy(h_hbm.at[pl.ds(r * EW, EW)], rowbuf)
            pltpu.sync_copy(rowbuf, acc.at[cstage], add=True)
            return 0

        lax.fori_loop(lo, hi, row, 0)

        pltpu.sync_copy(srcg.at[w], sstage)
        pltpu.sync_copy(dstg.at[w], dstage)

        def erowf(r, _):
            for g in range(EW // L):
                sv = sstage[r, pl.ds(g * L, L)]
                dv = dstage[r, pl.ds(g * L, L)]
                cs = plsc.load_gather(clu_v, [sv])
                cd = plsc.load_gather(clu_v, [dv])
                sstage[r, pl.ds(g * L, L)] = cs * n_clusters + cd
            return 0

        lax.fori_loop(0, erpt, erowf, 0)
        pltpu.sync_copy(sstage, eid_hbm.at[w])
        pltpu.sync_copy(hist, cnt_hbm.at[pl.ds(w * n_clusters, n_clusters)])
        plsc.subcore_barrier()

        def dblk(k, _):
            b = k * NS + sid

            @pl.when(b < nblk)
            def _():
                pltpu.sync_copy(acc.at[pl.ds(b * ZB, ZB)],
                                out_hbm.at[cid, pl.ds(b * ZB, ZB)])
            return 0

        lax.fori_loop(0, kmax, dblk, 0)

    return pl.kernel(
        body,
        out_type=[
            jax.ShapeDtypeStruct((NC, n_clusters, d), F32),
            jax.ShapeDtypeStruct((NW * n_clusters,), F32),
            jax.ShapeDtypeStruct((NW, n_edges // NW // EW, EW), I32),
        ],
        mesh=_mesh(),
        compiler_params=pltpu.CompilerParams(needs_layout_passes=False),
        scratch_types=[
            pltpu.VMEM((EW,), I32),
            pltpu.VMEM((EW, d), F32),
            pltpu.VMEM((n_clusters,), F32),
            pltpu.VMEM((n_nodes,), I32),
            pltpu.VMEM((erpt, EW), I32),
            pltpu.VMEM((erpt, EW), I32),
            pltpu.VMEM((ZB, d), F32),
            pltpu.VMEM_SHARED((n_clusters, d), F32),
        ],
    )


def _sc_ahist(n_clusters, n_edges):
    bins = n_clusters * n_clusters // NW
    chunk = n_edges // NW

    def body(eidf, out_hbm, estage, hist):
        w = _worker_id()
        lo_bin = w * bins
        zero = jnp.zeros((L,), F32)
        ones = jnp.ones((L,), F32)

        def zr(i, _):
            hist[pl.ds(i * L, L)] = zero
            return 0

        lax.fori_loop(0, bins // L, zr, 0)

        def ch(ci, _):
            pltpu.sync_copy(eidf.at[pl.ds(ci * chunk, chunk)], estage)

            def grp(g, _):
                e = estage[pl.ds(g * L, L)]
                loc = e - lo_bin
                m = (loc >= 0) & (loc < bins)
                locc = jnp.clip(loc, 0, bins - 1)
                plsc.addupdate_scatter(hist, [locc], ones, mask=m)
                return 0

            lax.fori_loop(0, chunk // L, grp, 0)
            return 0

        lax.fori_loop(0, NW, ch, 0)
        pltpu.sync_copy(hist, out_hbm.at[pl.ds(w * bins, bins)])

    return pl.kernel(
        body,
        out_type=jax.ShapeDtypeStruct((NW * bins,), F32),
        mesh=_mesh(),
        compiler_params=pltpu.CompilerParams(needs_layout_passes=False),
        scratch_types=[
            pltpu.VMEM((chunk,), I32),
            pltpu.VMEM((bins,), F32),
        ],
    )


def _tc_prep(n, d, h):
    def body(part, x, w, dinv_ref, y_ref):
        deg = lax.dot_general(part[...], jnp.ones((NW, 1), F32),
                              (((0,), (0,)), ((), ())),
                              preferred_element_type=F32) + 1.0
        dv = lax.rsqrt(deg)
        dinv_ref[...] = dv
        y_ref[...] = jnp.dot(x[...], w[...], preferred_element_type=F32) * dv

    return pl.pallas_call(
        body,
        out_shape=[jax.ShapeDtypeStruct((n, 1), F32),
                   jax.ShapeDtypeStruct((n, h), F32)],
    )


def _tc_layer2(n, h):
    def body(acc, y1, dinv, b1, w2, y2_ref):
        h1 = jnp.maximum(dinv[...] * (acc[0] + acc[1] + y1[...]) + b1[...], 0.0)
        y2_ref[...] = jnp.dot(h1, w2[...], preferred_element_type=F32) * dinv[...]

    return pl.pallas_call(
        body,
        out_shape=jax.ShapeDtypeStruct((n, h), F32),
    )


def _tc_kmeans(n, d):
    PAD = 512
    NP = n + PAD
    BLK = 256

    def body(acc, y2, dinv, b2, batch, h_ref, clu_ref, hpad, bpad, clpad):
        hh = dinv[...] * (acc[0] + acc[1] + y2[...]) + b2[...]
        h_ref[...] = hh
        hpad[0:n, :] = hh
        hpad[n:NP, :] = jnp.zeros((PAD, d), F32)
        bv = batch[...]
        bpad[0:n, :] = bv
        bpad[n:NP, :] = jnp.full((PAD, 1), NB, I32)

        def bloop(bi, _):
            start = jnp.sum((bv < bi).astype(I32))
            n_end = jnp.sum((bv < bi + 1).astype(I32))
            ws = pl.multiple_of((start // 8) * 8, 8)
            nblk = (n_end - ws + BLK - 1) // BLK
            c = hpad[pl.ds(start, KC), :]
            iota_k = lax.broadcasted_iota(I32, (BLK, KC), 1)
            onesd = jnp.ones((1, d), F32)
            onesb = jnp.ones((BLK, 1), F32)
            for it in range(KM_IT):
                csq = lax.dot_general(onesd, c * c, (((1,), (1,)), ((), ())),
                                      preferred_element_type=F32)
                last = it == KM_IT - 1

                def blk(j, carry):
                    r0 = ws + j * BLK
                    hb = hpad[pl.ds(r0, BLK), :]
                    valid = bpad[pl.ds(r0, BLK), :] == bi
                    xsq = jnp.sum(hb * hb, axis=1, keepdims=True)
                    d2 = xsq + csq - 2.0 * lax.dot_general(
                        hb, c, (((1,), (1,)), ((), ())),
                        preferred_element_type=F32)
                    mn = jnp.min(d2, axis=1, keepdims=True)
                    ass = jnp.min(jnp.where(d2 <= mn, iota_k, KC),
                                  axis=1, keepdims=True)
                    if last:
                        old = clpad[pl.ds(r0, BLK), :]
                        clpad[pl.ds(r0, BLK), :] = jnp.where(valid, ass + bi * KC, old)
                        return carry
                    oneh = ((iota_k == ass) & valid).astype(F32)
                    cs_, cn_ = carry
                    cs_ = cs_ + lax.dot_general(oneh, hb, (((0,), (0,)), ((), ())),
                                                preferred_element_type=F32)
                    cn_ = cn_ + lax.dot_general(oneh, onesb, (((0,), (0,)), ((), ())),
                                                preferred_element_type=F32)
                    return cs_, cn_

                if last:
                    lax.fori_loop(0, nblk, blk, 0)
                else:
                    cs_, cn_ = lax.fori_loop(
                        0, nblk, blk,
                        (jnp.zeros((KC, d), F32), jnp.zeros((KC, 1), F32)))
                    c = cs_ / jnp.maximum(cn_, 1.0)
            return 0

        lax.fori_loop(0, NB, bloop, 0)
        clu_ref[...] = clpad[0:n, :]

    return pl.pallas_call(
        body,
        out_shape=[jax.ShapeDtypeStruct((n, d), F32),
                   jax.ShapeDtypeStruct((n, 1), I32)],
        compiler_params=pltpu.CompilerParams(vmem_limit_bytes=63 * 1024 * 1024),
        scratch_shapes=[
            pltpu.VMEM((NP, d), F32),
            pltpu.VMEM((NP, 1), I32),
            pltpu.VMEM((NP, 1), I32),
        ],
    )


def _tc_final(d, out_dim):
    def body(A, cxs, cntp, wpr, bpr, w1, b1, w2, b2, wm1, bm1, wm2, bm2, out_ref):
        cnt = lax.dot_general(cntp[...], jnp.ones((NW, 1), F32),
                              (((0,), (0,)), ((), ())),
                              preferred_element_type=F32)
        cx = (cxs[0] + cxs[1]) / jnp.maximum(cnt, 1.0)
        cx = jnp.dot(cx, wpr[...], preferred_element_type=F32) + bpr[...]
        rr = lax.broadcasted_iota(I32, (CC, 1), 0)
        ccol = lax.broadcasted_iota(I32, (1, CC), 1)
        ind = jnp.where((A[...] > 0.0) & (rr != ccol), 1.0, 0.0)
        degc = lax.dot_general(ind, jnp.ones((CC, 1), F32),
                               (((0,), (0,)), ((), ())),
                               preferred_element_type=F32) + 1.0
        dinvc = lax.rsqrt(degc)

        def conv(xin, w, bias):
            z = jnp.dot(xin, w[...], preferred_element_type=F32) * dinvc
            t = lax.dot_general(ind, z, (((0,), (0,)), ((), ())),
                                preferred_element_type=F32)
            return dinvc * (t + z) + bias[...]

        h1 = jnp.maximum(conv(cx, w1, b1), 0.0)
        h2 = conv(h1, w2, b2)
        gi = lax.broadcasted_iota(I32, (NB, CC), 0)
        ci = lax.broadcasted_iota(I32, (NB, CC), 1)
        P = (ci // KC == gi).astype(F32)
        pooled = jnp.dot(P, h2, preferred_element_type=F32) / float(KC)
        hm = jnp.dot(pooled, wm1[...], preferred_element_type=F32) + bm1[...]
        hm = 0.5 * hm * (1.0 + lax.erf(hm * (2.0 ** -0.5)))
        out_ref[...] = jnp.dot(hm, wm2[...], preferred_element_type=F32) + bm2[...]

    return pl.pallas_call(
        body,
        out_shape=jax.ShapeDtypeStruct((NB, out_dim), F32),
    )


def kernel(x, edge_index, batch, W_g1, b_g1, W_g2, b_g2, W_proj, b_proj,
           W_p1, b_p1, W_p2, b_p2, W_m1, b_m1, W_m2, b_m2):
    n, d = x.shape
    e = edge_index.shape[1]
    h = W_g1.shape[1]
    out_dim = W_m2.shape[1]
    ept = e // NW

    src = edge_index[0]
    dst = edge_index[1]
    src3 = src.reshape(NW, ept // EW, EW)
    dst3 = dst.reshape(NW, ept // EW, EW)

    degp = _sc_deg(n, e)(dst3).reshape(NW, n)
    dinv, y1 = _tc_prep(n, d, h)(degp, x, W_g1)
    src4 = src.reshape(NW, ept // 50, 50)
    dst4 = dst.reshape(NW, ept // 50, 50)
    acc1 = _sc_msg(n, e, h)(y1, src4, dst4)
    y2 = _tc_layer2(n, h)(acc1, y1, dinv, b_g1.reshape(1, h), W_g2)
    acc2 = _sc_msg(n, e, h)(y2, src4, dst4)
    hfeat, clu2d = _tc_kmeans(n, h)(acc2, y2, dinv, b_g2.reshape(1, h),
                                    batch.reshape(n, 1))
    clu = clu2d.reshape(n)
    cxs, cntf, eid3 = _sc_stats(n, h, CC, e)(hfeat, clu, src3, dst3)
    cntp = cntf.reshape(NW, CC)
    acnt = _sc_ahist(CC, e)(eid3.reshape(e))
    A2 = acnt.reshape(CC, CC)
    return _tc_final(h, out_dim)(
        A2, cxs, cntp, W_proj, b_proj.reshape(1, h), W_p1, b_p1.reshape(1, h),
        W_p2, b_p2.reshape(1, h), W_m1, b_m1.reshape(1, h),
        W_m2, b_m2.reshape(1, out_dim))


# speedup vs baseline: 24.9154x; 1.2867x over previous
import jax
import jax.numpy as jnp
from jax import lax
from jax.experimental import pallas as pl
from jax.experimental.pallas import tpu as pltpu
from jax.experimental.pallas import tpu_sc as plsc

F32 = jnp.float32
I32 = jnp.int32

NC, NS, L = 2, 16, 16
NW = NC * NS

NB = 16
KC = 100
CC = NB * KC
KM_IT = 5

EW = 80


def _mesh():
    return plsc.VectorSubcoreMesh(core_axis_name="c", subcore_axis_name="s")


def _worker_id():
    return lax.axis_index("s") * NC + lax.axis_index("c")


def _sc_deg(n_nodes, n_edges):
    ept = n_edges // NW
    rpt = ept // EW

    def body(dst_hbm, out_hbm, stage, hist):
        w = _worker_id()
        zero = jnp.zeros((L,), F32)

        def zr(i, _):
            hist[pl.ds(i * L, L)] = zero
            return 0

        lax.fori_loop(0, n_nodes // L, zr, 0)
        pltpu.sync_copy(dst_hbm.at[w], stage)
        ones = jnp.ones((L,), F32)

        def row(r, _):
            for g in range(EW // L):
                idx = stage[r, pl.ds(g * L, L)]
                plsc.addupdate_scatter(hist, [idx], ones)
            return 0

        lax.fori_loop(0, rpt, row, 0)
        pltpu.sync_copy(hist, out_hbm.at[pl.ds(w * n_nodes, n_nodes)])

    return pl.kernel(
        body,
        out_type=jax.ShapeDtypeStruct((NW * n_nodes,), F32),
        mesh=_mesh(),
        compiler_params=pltpu.CompilerParams(needs_layout_passes=False),
        scratch_types=[
            pltpu.VMEM((rpt, EW), I32),
            pltpu.VMEM((n_nodes,), F32),
        ],
    )


def _sc_msg(n_nodes, n_edges, d):
    CH = 50
    SB = 40
    ept = n_edges // NW
    rpt = ept // CH
    nsb = rpt // SB
    ZB = 8
    nblk = n_nodes // ZB
    kmax = (nblk + NS - 1) // NS

    def body(y_hbm, src_hbm, dst_hbm, out_hbm,
             sidx, didx, rows0, rows1, rows2, rows3, zbuf, acc,
             sem0, sem1, sem2, sem3):
        cid = lax.axis_index("c")
        sid = lax.axis_index("s")
        w = sid * NC + cid
        zero = jnp.zeros((L,), F32)
        dl = d // L

        def zb(i, _):
            zbuf[i // dl, pl.ds((i % dl) * L, L)] = zero
            return 0

        lax.fori_loop(0, ZB * dl, zb, 0)

        def zblk(k, _):
            b = k * NS + sid

            @pl.when(b < nblk)
            def _():
                pltpu.sync_copy(zbuf, acc.at[pl.ds(b * ZB, ZB)])
            return 0

        lax.fori_loop(0, kmax, zblk, 0)
        plsc.subcore_barrier()

        bufs = (rows0, rows1, rows2, rows3)
        sems = (sem0, sem1, sem2, sem3)

        def sblk(k, _):
            pltpu.sync_copy(src_hbm.at[w, pl.ds(k * SB, SB)], sidx)
            pltpu.sync_copy(dst_hbm.at[w, pl.ds(k * SB, SB)], didx)
            for j in range(4):
                pltpu.async_copy(y_hbm.at[sidx.at[j]], bufs[j], sems[j])

            def quad(i, _):
                for j in range(4):
                    g = i * 4 + j
                    pltpu.make_async_copy(y_hbm.at[sidx.at[g]], bufs[j],
                                          sems[j]).wait()
                    pltpu.sync_copy(bufs[j], acc.at[didx.at[g]], add=True)

                    @pl.when(g + 4 < SB)
                    def _():
                        pltpu.async_copy(y_hbm.at[sidx.at[g + 4]], bufs[j],
                                         sems[j])
                return 0

            lax.fori_loop(0, SB // 4, quad, 0)
            return 0

        lax.fori_loop(0, nsb, sblk, 0)
        plsc.subcore_barrier()

        def dblk(k, _):
            b = k * NS + sid

            @pl.when(b < nblk)
            def _():
                pltpu.sync_copy(acc.at[pl.ds(b * ZB, ZB)],
                                out_hbm.at[cid, pl.ds(b * ZB, ZB)])
            return 0

        lax.fori_loop(0, kmax, dblk, 0)

    return pl.kernel(
        body,
        out_type=jax.ShapeDtypeStruct((NC, n_nodes, d), F32),
        mesh=_mesh(),
        compiler_params=pltpu.CompilerParams(needs_layout_passes=False),
        scratch_types=[
            pltpu.VMEM((SB, CH), I32),
            pltpu.VMEM((SB, CH), I32),
            pltpu.VMEM((CH, d), F32),
            pltpu.VMEM((CH, d), F32),
            pltpu.VMEM((CH, d), F32),
            pltpu.VMEM((CH, d), F32),
            pltpu.VMEM((ZB, d), F32),
            pltpu.VMEM_SHARED((n_nodes, d), F32),
            pltpu.SemaphoreType.DMA,
            pltpu.SemaphoreType.DMA,
            pltpu.SemaphoreType.DMA,
            pltpu.SemaphoreType.DMA,
        ],
    )


def _sc_stats(n_nodes, d, n_clusters, n_edges):
    nrow = n_nodes // EW
    ept = n_edges // NW
    erpt = ept // EW
    ZB = 80
    nblk = n_clusters // ZB
    kmax = (nblk + NS - 1) // NS

    def body(h_hbm, clu_hbm, srcg, dstg,
             out_hbm, cnt_hbm, eid_hbm,
             cstage, rowbuf, hist, clu_v, sstage, dstage, zbuf, acc):
        cid = lax.axis_index("c")
        sid = lax.axis_index("s")
        w = sid * NC + cid
        zero = jnp.zeros((L,), F32)
        dl = d // L

        def zb(i, _):
            zbuf[i // dl, pl.ds((i % dl) * L, L)] = zero
            return 0

        lax.fori_loop(0, ZB * dl, zb, 0)

        def zblk(k, _):
            b = k * NS + sid

            @pl.when(b < nblk)
            def _():
                pltpu.sync_copy(zbuf, acc.at[pl.ds(b * ZB, ZB)])
            return 0

        lax.fori_loop(0, kmax, zblk, 0)

        def zh(i, _):
            hist[pl.ds(i * L, L)] = zero
            return 0

        lax.fori_loop(0, n_clusters // L, zh, 0)
        pltpu.sync_copy(clu_hbm, clu_v)
        plsc.subcore_barrier()

        ones = jnp.ones((L,), F32)
        lo = (w * nrow) // NW
        hi = ((w + 1) * nrow) // NW

        def row(r, _):
            for g in range(EW // L):
                v = clu_v[pl.ds(r * EW + g * L, L)]
                cstage[pl.ds(g * L, L)] = v
                plsc.addupdate_scatter(hist, [v], ones)
            pltpu.sync_copy(h_hbm.at[pl.ds(r * EW, EW)], rowbuf)
            pltpu.sync_copy(rowbuf, acc.at[cstage], add=True)
            return 0

        lax.fori_loop(lo, hi, row, 0)

        pltpu.sync_copy(srcg.at[w], sstage)
        pltpu.sync_copy(dstg.at[w], dstage)

        def erowf(r, _):
            for g in range(EW // L):
                sv = sstage[r, pl.ds(g * L, L)]
                dv = dstage[r, pl.ds(g * L, L)]
                cs = plsc.load_gather(clu_v, [sv])
                cd = plsc.load_gather(clu_v, [dv])
                sstage[r, pl.ds(g * L, L)] = cs * n_clusters + cd
            return 0

        lax.fori_loop(0, erpt, erowf, 0)
        pltpu.sync_copy(sstage, eid_hbm.at[w])
        pltpu.sync_copy(hist, cnt_hbm.at[pl.ds(w * n_clusters, n_clusters)])
        plsc.subcore_barrier()

        def dblk(k, _):
            b = k * NS + sid

            @pl.when(b < nblk)
            def _():
                pltpu.sync_copy(acc.at[pl.ds(b * ZB, ZB)],
                                out_hbm.at[cid, pl.ds(b * ZB, ZB)])
            return 0

        lax.fori_loop(0, kmax, dblk, 0)

    return pl.kernel(
        body,
        out_type=[
            jax.ShapeDtypeStruct((NC, n_clusters, d), F32),
            jax.ShapeDtypeStruct((NW * n_clusters,), F32),
            jax.ShapeDtypeStruct((NW, n_edges // NW // EW, EW), I32),
        ],
        mesh=_mesh(),
        compiler_params=pltpu.CompilerParams(needs_layout_passes=False),
        scratch_types=[
            pltpu.VMEM((EW,), I32),
            pltpu.VMEM((EW, d), F32),
            pltpu.VMEM((n_clusters,), F32),
            pltpu.VMEM((n_nodes,), I32),
            pltpu.VMEM((erpt, EW), I32),
            pltpu.VMEM((erpt, EW), I32),
            pltpu.VMEM((ZB, d), F32),
            pltpu.VMEM_SHARED((n_clusters, d), F32),
        ],
    )


def _sc_ahist(n_clusters, n_edges):
    bins = n_clusters * n_clusters // NW
    chunk = n_edges // NW

    def body(eidf, out_hbm, estage, hist):
        w = _worker_id()
        lo_bin = w * bins
        zero = jnp.zeros((L,), F32)
        ones = jnp.ones((L,), F32)

        def zr(i, _):
            hist[pl.ds(i * L, L)] = zero
            return 0

        lax.fori_loop(0, bins // L, zr, 0)

        def ch(ci, _):
            pltpu.sync_copy(eidf.at[pl.ds(ci * chunk, chunk)], estage)

            def grp(g, _):
                e = estage[pl.ds(g * L, L)]
                loc = e - lo_bin
                m = (loc >= 0) & (loc < bins)
                locc = jnp.clip(loc, 0, bins - 1)
                plsc.addupdate_scatter(hist, [locc], ones, mask=m)
                return 0

            lax.fori_loop(0, chunk // L, grp, 0)
            return 0

        lax.fori_loop(0, NW, ch, 0)
        pltpu.sync_copy(hist, out_hbm.at[pl.ds(w * bins, bins)])

    return pl.kernel(
        body,
        out_type=jax.ShapeDtypeStruct((NW * bins,), F32),
        mesh=_mesh(),
        compiler_params=pltpu.CompilerParams(needs_layout_passes=False),
        scratch_types=[
            pltpu.VMEM((chunk,), I32),
            pltpu.VMEM((bins,), F32),
        ],
    )


def _tc_prep(n, d, h):
    def body(part, x, w, dinv_ref, y_ref):
        deg = lax.dot_general(part[...], jnp.ones((NW, 1), F32),
                              (((0,), (0,)), ((), ())),
                              preferred_element_type=F32) + 1.0
        dv = lax.rsqrt(deg)
        dinv_ref[...] = dv
        y_ref[...] = jnp.dot(x[...], w[...], preferred_element_type=F32) * dv

    return pl.pallas_call(
        body,
        out_shape=[jax.ShapeDtypeStruct((n, 1), F32),
                   jax.ShapeDtypeStruct((n, h), F32)],
    )


def _tc_layer2(n, h):
    def body(acc, y1, dinv, b1, w2, y2_ref):
        h1 = jnp.maximum(dinv[...] * (acc[0] + acc[1] + y1[...]) + b1[...], 0.0)
        y2_ref[...] = jnp.dot(h1, w2[...], preferred_element_type=F32) * dinv[...]

    return pl.pallas_call(
        body,
        out_shape=jax.ShapeDtypeStruct((n, h), F32),
    )


def _tc_kmeans(n, d):
    PAD = 512
    NP = n + PAD
    BLK = 256

    def body(acc, y2, dinv, b2, batch, h_ref, clu_ref, hpad, bpad, clpad):
        hh = dinv[...] * (acc[0] + acc[1] + y2[...]) + b2[...]
        h_ref[...] = hh
        hpad[0:n, :] = hh
        hpad[n:NP, :] = jnp.zeros((PAD, d), F32)
        bv = batch[...]
        bpad[0:n, :] = bv
        bpad[n:NP, :] = jnp.full((PAD, 1), NB, I32)

        def bloop(bi, _):
            start = jnp.sum((bv < bi).astype(I32))
            n_end = jnp.sum((bv < bi + 1).astype(I32))
            ws = pl.multiple_of((start // 8) * 8, 8)
            nblk = (n_end - ws + BLK - 1) // BLK
            c = hpad[pl.ds(start, KC), :]
            iota_k = lax.broadcasted_iota(I32, (BLK, KC), 1)
            onesd = jnp.ones((1, d), F32)
            onesb = jnp.ones((BLK, 1), F32)
            for it in range(KM_IT):
                csq = lax.dot_general(onesd, c * c, (((1,), (1,)), ((), ())),
                                      preferred_element_type=F32)
                last = it == KM_IT - 1

                def blk(j, carry):
                    r0 = ws + j * BLK
                    hb = hpad[pl.ds(r0, BLK), :]
                    valid = bpad[pl.ds(r0, BLK), :] == bi
                    xsq = jnp.sum(hb * hb, axis=1, keepdims=True)
                    d2 = xsq + csq - 2.0 * lax.dot_general(
                        hb, c, (((1,), (1,)), ((), ())),
                        preferred_element_type=F32)
                    mn = jnp.min(d2, axis=1, keepdims=True)
                    ass = jnp.min(jnp.where(d2 <= mn, iota_k, KC),
                                  axis=1, keepdims=True)
                    if last:
                        old = clpad[pl.ds(r0, BLK), :]
                        clpad[pl.ds(r0, BLK), :] = jnp.where(valid, ass + bi * KC, old)
                        return carry
                    oneh = ((iota_k == ass) & valid).astype(F32)
                    cs_, cn_ = carry
                    cs_ = cs_ + lax.dot_general(oneh, hb, (((0,), (0,)), ((), ())),
                                                preferred_element_type=F32)
                    cn_ = cn_ + lax.dot_general(oneh, onesb, (((0,), (0,)), ((), ())),
                                                preferred_element_type=F32)
                    return cs_, cn_

                if last:
                    lax.fori_loop(0, nblk, blk, 0)
                else:
                    cs_, cn_ = lax.fori_loop(
                        0, nblk, blk,
                        (jnp.zeros((KC, d), F32), jnp.zeros((KC, 1), F32)))
                    c = cs_ / jnp.maximum(cn_, 1.0)
            return 0

        lax.fori_loop(0, NB, bloop, 0)
        clu_ref[...] = clpad[0:n, :]

    return pl.pallas_call(
        body,
        out_shape=[jax.ShapeDtypeStruct((n, d), F32),
                   jax.ShapeDtypeStruct((n, 1), I32)],
        compiler_params=pltpu.CompilerParams(vmem_limit_bytes=63 * 1024 * 1024),
        scratch_shapes=[
            pltpu.VMEM((NP, d), F32),
            pltpu.VMEM((NP, 1), I32),
            pltpu.VMEM((NP, 1), I32),
        ],
    )


def _tc_final(d, out_dim):
    def body(A, cxs, cntp, wpr, bpr, w1, b1, w2, b2, wm1, bm1, wm2, bm2, out_ref):
        cnt = lax.dot_general(cntp[...], jnp.ones((NW, 1), F32),
                              (((0,), (0,)), ((), ())),
                              preferred_element_type=F32)
        cx = (cxs[0] + cxs[1]) / jnp.maximum(cnt, 1.0)
        cx = jnp.dot(cx, wpr[...], preferred_element_type=F32) + bpr[...]
        rr = lax.broadcasted_iota(I32, (CC, 1), 0)
        ccol = lax.broadcasted_iota(I32, (1, CC), 1)
        ind = jnp.where((A[...] > 0.0) & (rr != ccol), 1.0, 0.0)
        degc = lax.dot_general(ind, jnp.ones((CC, 1), F32),
                               (((0,), (0,)), ((), ())),
                               preferred_element_type=F32) + 1.0
        dinvc = lax.rsqrt(degc)

        def conv(xin, w, bias):
            z = jnp.dot(xin, w[...], preferred_element_type=F32) * dinvc
            t = lax.dot_general(ind, z, (((0,), (0,)), ((), ())),
                                preferred_element_type=F32)
            return dinvc * (t + z) + bias[...]

        h1 = jnp.maximum(conv(cx, w1, b1), 0.0)
        h2 = conv(h1, w2, b2)
        gi = lax.broadcasted_iota(I32, (NB, CC), 0)
        ci = lax.broadcasted_iota(I32, (NB, CC), 1)
        P = (ci // KC == gi).astype(F32)
        pooled = jnp.dot(P, h2, preferred_element_type=F32) / float(KC)
        hm = jnp.dot(pooled, wm1[...], preferred_element_type=F32) + bm1[...]
        hm = 0.5 * hm * (1.0 + lax.erf(hm * (2.0 ** -0.5)))
        out_ref[...] = jnp.dot(hm, wm2[...], preferred_element_type=F32) + bm2[...]

    return pl.pallas_call(
        body,
        out_shape=jax.ShapeDtypeStruct((NB, out_dim), F32),
    )


def kernel(x, edge_index, batch, W_g1, b_g1, W_g2, b_g2, W_proj, b_proj,
           W_p1, b_p1, W_p2, b_p2, W_m1, b_m1, W_m2, b_m2):
    n, d = x.shape
    e = edge_index.shape[1]
    h = W_g1.shape[1]
    out_dim = W_m2.shape[1]
    ept = e // NW

    src = edge_index[0]
    dst = edge_index[1]
    src3 = src.reshape(NW, ept // EW, EW)
    dst3 = dst.reshape(NW, ept // EW, EW)

    degp = _sc_deg(n, e)(dst3).reshape(NW, n)
    dinv, y1 = _tc_prep(n, d, h)(degp, x, W_g1)
    src4 = src.reshape(NW, ept // 50, 50)
    dst4 = dst.reshape(NW, ept // 50, 50)
    acc1 = _sc_msg(n, e, h)(y1, src4, dst4)
    y2 = _tc_layer2(n, h)(acc1, y1, dinv, b_g1.reshape(1, h), W_g2)
    acc2 = _sc_msg(n, e, h)(y2, src4, dst4)
    hfeat, clu2d = _tc_kmeans(n, h)(acc2, y2, dinv, b_g2.reshape(1, h),
                                    batch.reshape(n, 1))
    clu = clu2d.reshape(n)
    cxs, cntf, eid3 = _sc_stats(n, h, CC, e)(hfeat, clu, src3, dst3)
    cntp = cntf.reshape(NW, CC)
    acnt = _sc_ahist(CC, e)(eid3.reshape(e))
    A2 = acnt.reshape(CC, CC)
    return _tc_final(h, out_dim)(
        A2, cxs, cntp, W_proj, b_proj.reshape(1, h), W_p1, b_p1.reshape(1, h),
        W_p2, b_p2.reshape(1, h), W_m1, b_m1.reshape(1, h),
        W_m2, b_m2.reshape(1, out_dim))


# speedup vs baseline: 25.3918x; 1.0191x over previous
import jax
import jax.numpy as jnp
from jax import lax
from jax.experimental import pallas as pl
from jax.experimental.pallas import tpu as pltpu
from jax.experimental.pallas import tpu_sc as plsc

F32 = jnp.float32
I32 = jnp.int32

NC, NS, L = 2, 16, 16
NW = NC * NS

NB = 16
KC = 100
CC = NB * KC
KM_IT = 5

EW = 80


def _mesh():
    return plsc.VectorSubcoreMesh(core_axis_name="c", subcore_axis_name="s")


def _worker_id():
    return lax.axis_index("s") * NC + lax.axis_index("c")


def _sc_deg(n_nodes, n_edges):
    ept = n_edges // NW
    rpt = ept // EW

    def body(dst_hbm, out_hbm, stage, hist):
        w = _worker_id()
        zero = jnp.zeros((L,), F32)

        def zr(i, _):
            hist[pl.ds(i * L, L)] = zero
            return 0

        lax.fori_loop(0, n_nodes // L, zr, 0)
        pltpu.sync_copy(dst_hbm.at[w], stage)
        ones = jnp.ones((L,), F32)

        def row(r, _):
            for g in range(EW // L):
                idx = stage[r, pl.ds(g * L, L)]
                plsc.addupdate_scatter(hist, [idx], ones)
            return 0

        lax.fori_loop(0, rpt, row, 0)
        pltpu.sync_copy(hist, out_hbm.at[pl.ds(w * n_nodes, n_nodes)])

    return pl.kernel(
        body,
        out_type=jax.ShapeDtypeStruct((NW * n_nodes,), F32),
        mesh=_mesh(),
        compiler_params=pltpu.CompilerParams(needs_layout_passes=False),
        scratch_types=[
            pltpu.VMEM((rpt, EW), I32),
            pltpu.VMEM((n_nodes,), F32),
        ],
    )


def _sc_msg(n_nodes, n_edges, d):
    CH = 50
    SB = 40
    ept = n_edges // NW
    rpt = ept // CH
    nsb = rpt // SB
    ZB = 8
    nblk = n_nodes // ZB
    kmax = (nblk + NS - 1) // NS

    def body(y_hbm, src_hbm, dst_hbm, out_hbm,
             sidx, didx, rows0, rows1, rows2, rows3, zbuf, acc,
             sem0, sem1, sem2, sem3):
        cid = lax.axis_index("c")
        sid = lax.axis_index("s")
        w = sid * NC + cid
        zero = jnp.zeros((L,), F32)
        dl = d // L

        def zb(i, _):
            zbuf[i // dl, pl.ds((i % dl) * L, L)] = zero
            return 0

        lax.fori_loop(0, ZB * dl, zb, 0)

        def zblk(k, _):
            b = k * NS + sid

            @pl.when(b < nblk)
            def _():
                pltpu.sync_copy(zbuf, acc.at[pl.ds(b * ZB, ZB)])
            return 0

        lax.fori_loop(0, kmax, zblk, 0)
        plsc.subcore_barrier()

        bufs = (rows0, rows1, rows2, rows3)
        sems = (sem0, sem1, sem2, sem3)

        def sblk(k, _):
            pltpu.sync_copy(src_hbm.at[w, pl.ds(k * SB, SB)], sidx)
            pltpu.sync_copy(dst_hbm.at[w, pl.ds(k * SB, SB)], didx)
            for j in range(4):
                pltpu.async_copy(y_hbm.at[sidx.at[j]], bufs[j], sems[j])

            def quad(i, _):
                for j in range(4):
                    g = i * 4 + j
                    pltpu.make_async_copy(y_hbm.at[sidx.at[g]], bufs[j],
                                          sems[j]).wait()
                    pltpu.sync_copy(bufs[j], acc.at[didx.at[g]], add=True)

                    @pl.when(g + 4 < SB)
                    def _():
                        pltpu.async_copy(y_hbm.at[sidx.at[g + 4]], bufs[j],
                                         sems[j])
                return 0

            lax.fori_loop(0, SB // 4, quad, 0)
            return 0

        lax.fori_loop(0, nsb, sblk, 0)
        plsc.subcore_barrier()

        def dblk(k, _):
            b = k * NS + sid

            @pl.when(b < nblk)
            def _():
                pltpu.sync_copy(acc.at[pl.ds(b * ZB, ZB)],
                                out_hbm.at[cid, pl.ds(b * ZB, ZB)])
            return 0

        lax.fori_loop(0, kmax, dblk, 0)

    return pl.kernel(
        body,
        out_type=jax.ShapeDtypeStruct((NC, n_nodes, d), F32),
        mesh=_mesh(),
        compiler_params=pltpu.CompilerParams(needs_layout_passes=False),
        scratch_types=[
            pltpu.VMEM((SB, CH), I32),
            pltpu.VMEM((SB, CH), I32),
            pltpu.VMEM((CH, d), F32),
            pltpu.VMEM((CH, d), F32),
            pltpu.VMEM((CH, d), F32),
            pltpu.VMEM((CH, d), F32),
            pltpu.VMEM((ZB, d), F32),
            pltpu.VMEM_SHARED((n_nodes, d), F32),
            pltpu.SemaphoreType.DMA,
            pltpu.SemaphoreType.DMA,
            pltpu.SemaphoreType.DMA,
            pltpu.SemaphoreType.DMA,
        ],
    )


def _sc_stats(n_nodes, d, n_clusters, n_edges):
    nrow = n_nodes // EW
    ept = n_edges // NW
    erpt = ept // EW
    ZB = 80
    nblk = n_clusters // ZB
    kmax = (nblk + NS - 1) // NS

    def body(h_hbm, clu_hbm, srcg, dstg,
             out_hbm, cnt_hbm, eid_hbm,
             cstage, rowbuf, hist, clu_v, sstage, dstage, zbuf, acc):
        cid = lax.axis_index("c")
        sid = lax.axis_index("s")
        w = sid * NC + cid
        zero = jnp.zeros((L,), F32)
        dl = d // L

        def zb(i, _):
            zbuf[i // dl, pl.ds((i % dl) * L, L)] = zero
            return 0

        lax.fori_loop(0, ZB * dl, zb, 0)

        def zblk(k, _):
            b = k * NS + sid

            @pl.when(b < nblk)
            def _():
                pltpu.sync_copy(zbuf, acc.at[pl.ds(b * ZB, ZB)])
            return 0

        lax.fori_loop(0, kmax, zblk, 0)

        def zh(i, _):
            hist[pl.ds(i * L, L)] = zero
            return 0

        lax.fori_loop(0, n_clusters // L, zh, 0)
        pltpu.sync_copy(clu_hbm, clu_v)
        plsc.subcore_barrier()

        ones = jnp.ones((L,), F32)
        lo = (w * nrow) // NW
        hi = ((w + 1) * nrow) // NW

        def row(r, _):
            for g in range(EW // L):
                v = clu_v[pl.ds(r * EW + g * L, L)]
                cstage[pl.ds(g * L, L)] = v
                plsc.addupdate_scatter(hist, [v], ones)
            pltpu.sync_copy(h_hbm.at[pl.ds(r * EW, EW)], rowbuf)
            pltpu.sync_copy(rowbuf, acc.at[cstage], add=True)
            return 0

        lax.fori_loop(lo, hi, row, 0)

        pltpu.sync_copy(srcg.at[w], sstage)
        pltpu.sync_copy(dstg.at[w], dstage)

        def erowf(r, _):
            for g in range(EW // L):
                sv = sstage[r, pl.ds(g * L, L)]
                dv = dstage[r, pl.ds(g * L, L)]
                cs = plsc.load_gather(clu_v, [sv])
                cd = plsc.load_gather(clu_v, [dv])
                sstage[r, pl.ds(g * L, L)] = cs * n_clusters + cd
            return 0

        lax.fori_loop(0, erpt, erowf, 0)
        pltpu.sync_copy(sstage, eid_hbm.at[w])
        pltpu.sync_copy(hist, cnt_hbm.at[pl.ds(w * n_clusters, n_clusters)])
        plsc.subcore_barrier()

        def dblk(k, _):
            b = k * NS + sid

            @pl.when(b < nblk)
            def _():
                pltpu.sync_copy(acc.at[pl.ds(b * ZB, ZB)],
                                out_hbm.at[cid, pl.ds(b * ZB, ZB)])
            return 0

        lax.fori_loop(0, kmax, dblk, 0)

    return pl.kernel(
        body,
        out_type=[
            jax.ShapeDtypeStruct((NC, n_clusters, d), F32),
            jax.ShapeDtypeStruct((NW * n_clusters,), F32),
            jax.ShapeDtypeStruct((NW, n_edges // NW // EW, EW), I32),
        ],
        mesh=_mesh(),
        compiler_params=pltpu.CompilerParams(needs_layout_passes=False),
        scratch_types=[
            pltpu.VMEM((EW,), I32),
            pltpu.VMEM((EW, d), F32),
            pltpu.VMEM((n_clusters,), F32),
            pltpu.VMEM((n_nodes,), I32),
            pltpu.VMEM((erpt, EW), I32),
            pltpu.VMEM((erpt, EW), I32),
            pltpu.VMEM((ZB, d), F32),
            pltpu.VMEM_SHARED((n_clusters, d), F32),
        ],
    )


def _sc_ahist(n_clusters, n_edges):
    bins = n_clusters * n_clusters // NW
    chunk = n_edges // NW

    def body(eidf, out_hbm, estage, hist):
        w = _worker_id()
        lo_bin = w * bins
        zero = jnp.zeros((L,), F32)
        ones = jnp.ones((L,), F32)

        def zr(i, _):
            hist[pl.ds(i * L, L)] = zero
            return 0

        lax.fori_loop(0, bins // L, zr, 0)

        def ch(ci, _):
            pltpu.sync_copy(eidf.at[pl.ds(ci * chunk, chunk)], estage)

            def grp(g, _):
                for u in range(5):
                    e = estage[pl.ds(g * (5 * L) + u * L, L)]
                    loc = e - lo_bin
                    m = loc.astype(jnp.uint32) < jnp.uint32(bins)
                    locc = jnp.clip(loc, 0, bins - 1)
                    plsc.addupdate_scatter(hist, [locc], ones, mask=m)
                return 0

            lax.fori_loop(0, chunk // (5 * L), grp, 0)
            return 0

        lax.fori_loop(0, NW, ch, 0)
        pltpu.sync_copy(hist, out_hbm.at[pl.ds(w * bins, bins)])

    return pl.kernel(
        body,
        out_type=jax.ShapeDtypeStruct((NW * bins,), F32),
        mesh=_mesh(),
        compiler_params=pltpu.CompilerParams(needs_layout_passes=False),
        scratch_types=[
            pltpu.VMEM((chunk,), I32),
            pltpu.VMEM((bins,), F32),
        ],
    )


def _tc_prep(n, d, h):
    def body(part, x, w, dinv_ref, y_ref):
        deg = lax.dot_general(part[...], jnp.ones((NW, 1), F32),
                              (((0,), (0,)), ((), ())),
                              preferred_element_type=F32) + 1.0
        dv = lax.rsqrt(deg)
        dinv_ref[...] = dv
        y_ref[...] = jnp.dot(x[...], w[...], preferred_element_type=F32) * dv

    return pl.pallas_call(
        body,
        out_shape=[jax.ShapeDtypeStruct((n, 1), F32),
                   jax.ShapeDtypeStruct((n, h), F32)],
    )


def _tc_layer2(n, h):
    def body(acc, y1, dinv, b1, w2, y2_ref):
        h1 = jnp.maximum(dinv[...] * (acc[0] + acc[1] + y1[...]) + b1[...], 0.0)
        y2_ref[...] = jnp.dot(h1, w2[...], preferred_element_type=F32) * dinv[...]

    return pl.pallas_call(
        body,
        out_shape=jax.ShapeDtypeStruct((n, h), F32),
    )


def _tc_kmeans(n, d):
    PAD = 512
    NP = n + PAD
    BLK = 256

    def body(acc, y2, dinv, b2, batch, h_ref, clu_ref, hpad, bpad, clpad):
        hh = dinv[...] * (acc[0] + acc[1] + y2[...]) + b2[...]
        h_ref[...] = hh
        hpad[0:n, :] = hh
        hpad[n:NP, :] = jnp.zeros((PAD, d), F32)
        bv = batch[...]
        bpad[0:n, :] = bv
        bpad[n:NP, :] = jnp.full((PAD, 1), NB, I32)

        def bloop(bi, _):
            start = jnp.sum((bv < bi).astype(I32))
            n_end = jnp.sum((bv < bi + 1).astype(I32))
            ws = pl.multiple_of((start // 8) * 8, 8)
            nblk = (n_end - ws + BLK - 1) // BLK
            c = hpad[pl.ds(start, KC), :]
            iota_k = lax.broadcasted_iota(I32, (BLK, KC), 1)
            onesd = jnp.ones((1, d), F32)
            onesb = jnp.ones((BLK, 1), F32)
            for it in range(KM_IT):
                csq = lax.dot_general(onesd, c * c, (((1,), (1,)), ((), ())),
                                      preferred_element_type=F32)
                last = it == KM_IT - 1

                def blk(j, carry):
                    r0 = ws + j * BLK
                    hb = hpad[pl.ds(r0, BLK), :]
                    valid = bpad[pl.ds(r0, BLK), :] == bi
                    xsq = jnp.sum(hb * hb, axis=1, keepdims=True)
                    d2 = xsq + csq - 2.0 * lax.dot_general(
                        hb, c, (((1,), (1,)), ((), ())),
                        preferred_element_type=F32)
                    mn = jnp.min(d2, axis=1, keepdims=True)
                    ass = jnp.min(jnp.where(d2 <= mn, iota_k, KC),
                                  axis=1, keepdims=True)
                    if last:
                        old = clpad[pl.ds(r0, BLK), :]
                        clpad[pl.ds(r0, BLK), :] = jnp.where(valid, ass + bi * KC, old)
                        return carry
                    oneh = ((iota_k == ass) & valid).astype(F32)
                    cs_, cn_ = carry
                    cs_ = cs_ + lax.dot_general(oneh, hb, (((0,), (0,)), ((), ())),
                                                preferred_element_type=F32)
                    cn_ = cn_ + lax.dot_general(oneh, onesb, (((0,), (0,)), ((), ())),
                                                preferred_element_type=F32)
                    return cs_, cn_

                if last:
                    lax.fori_loop(0, nblk, blk, 0)
                else:
                    cs_, cn_ = lax.fori_loop(
                        0, nblk, blk,
                        (jnp.zeros((KC, d), F32), jnp.zeros((KC, 1), F32)))
                    c = cs_ / jnp.maximum(cn_, 1.0)
            return 0

        lax.fori_loop(0, NB, bloop, 0)
        clu_ref[...] = clpad[0:n, :]

    return pl.pallas_call(
        body,
        out_shape=[jax.ShapeDtypeStruct((n, d), F32),
                   jax.ShapeDtypeStruct((n, 1), I32)],
        compiler_params=pltpu.CompilerParams(vmem_limit_bytes=63 * 1024 * 1024),
        scratch_shapes=[
            pltpu.VMEM((NP, d), F32),
            pltpu.VMEM((NP, 1), I32),
            pltpu.VMEM((NP, 1), I32),
        ],
    )


def _tc_final(d, out_dim):
    def body(A, cxs, cntp, wpr, bpr, w1, b1, w2, b2, wm1, bm1, wm2, bm2, out_ref):
        cnt = lax.dot_general(cntp[...], jnp.ones((NW, 1), F32),
                              (((0,), (0,)), ((), ())),
                              preferred_element_type=F32)
        cx = (cxs[0] + cxs[1]) / jnp.maximum(cnt, 1.0)
        cx = jnp.dot(cx, wpr[...], preferred_element_type=F32) + bpr[...]
        rr = lax.broadcasted_iota(I32, (CC, 1), 0)
        ccol = lax.broadcasted_iota(I32, (1, CC), 1)
        ind = jnp.where((A[...] > 0.0) & (rr != ccol), 1.0, 0.0)
        degc = lax.dot_general(ind, jnp.ones((CC, 1), F32),
                               (((0,), (0,)), ((), ())),
                               preferred_element_type=F32) + 1.0
        dinvc = lax.rsqrt(degc)

        def conv(xin, w, bias):
            z = jnp.dot(xin, w[...], preferred_element_type=F32) * dinvc
            t = lax.dot_general(ind, z, (((0,), (0,)), ((), ())),
                                preferred_element_type=F32)
            return dinvc * (t + z) + bias[...]

        h1 = jnp.maximum(conv(cx, w1, b1), 0.0)
        h2 = conv(h1, w2, b2)
        gi = lax.broadcasted_iota(I32, (NB, CC), 0)
        ci = lax.broadcasted_iota(I32, (NB, CC), 1)
        P = (ci // KC == gi).astype(F32)
        pooled = jnp.dot(P, h2, preferred_element_type=F32) / float(KC)
        hm = jnp.dot(pooled, wm1[...], preferred_element_type=F32) + bm1[...]
        hm = 0.5 * hm * (1.0 + lax.erf(hm * (2.0 ** -0.5)))
        out_ref[...] = jnp.dot(hm, wm2[...], preferred_element_type=F32) + bm2[...]

    return pl.pallas_call(
        body,
        out_shape=jax.ShapeDtypeStruct((NB, out_dim), F32),
    )


def kernel(x, edge_index, batch, W_g1, b_g1, W_g2, b_g2, W_proj, b_proj,
           W_p1, b_p1, W_p2, b_p2, W_m1, b_m1, W_m2, b_m2):
    n, d = x.shape
    e = edge_index.shape[1]
    h = W_g1.shape[1]
    out_dim = W_m2.shape[1]
    ept = e // NW

    src = edge_index[0]
    dst = edge_index[1]
    src3 = src.reshape(NW, ept // EW, EW)
    dst3 = dst.reshape(NW, ept // EW, EW)

    degp = _sc_deg(n, e)(dst3).reshape(NW, n)
    dinv, y1 = _tc_prep(n, d, h)(degp, x, W_g1)
    src4 = src.reshape(NW, ept // 50, 50)
    dst4 = dst.reshape(NW, ept // 50, 50)
    acc1 = _sc_msg(n, e, h)(y1, src4, dst4)
    y2 = _tc_layer2(n, h)(acc1, y1, dinv, b_g1.reshape(1, h), W_g2)
    acc2 = _sc_msg(n, e, h)(y2, src4, dst4)
    hfeat, clu2d = _tc_kmeans(n, h)(acc2, y2, dinv, b_g2.reshape(1, h),
                                    batch.reshape(n, 1))
    clu = clu2d.reshape(n)
    cxs, cntf, eid3 = _sc_stats(n, h, CC, e)(hfeat, clu, src3, dst3)
    cntp = cntf.reshape(NW, CC)
    acnt = _sc_ahist(CC, e)(eid3.reshape(e))
    A2 = acnt.reshape(CC, CC)
    return _tc_final(h, out_dim)(
        A2, cxs, cntp, W_proj, b_proj.reshape(1, h), W_p1, b_p1.reshape(1, h),
        W_p2, b_p2.reshape(1, h), W_m1, b_m1.reshape(1, h),
        W_m2, b_m2.reshape(1, out_dim))


# speedup vs baseline: 27.7382x; 1.0924x over previous
import jax
import jax.numpy as jnp
from jax import lax
from jax.experimental import pallas as pl
from jax.experimental.pallas import tpu as pltpu
from jax.experimental.pallas import tpu_sc as plsc

F32 = jnp.float32
I32 = jnp.int32

NC, NS, L = 2, 16, 16
NW = NC * NS

NB = 16
KC = 100
CC = NB * KC
KM_IT = 5

EW = 80


def _mesh():
    return plsc.VectorSubcoreMesh(core_axis_name="c", subcore_axis_name="s")


def _worker_id():
    return lax.axis_index("s") * NC + lax.axis_index("c")


def _sc_deg(n_nodes, n_edges):
    ept = n_edges // NW
    rpt = ept // EW

    def body(dst_hbm, out_hbm, stage, hist):
        w = _worker_id()
        zero = jnp.zeros((L,), F32)

        def zr(i, _):
            hist[pl.ds(i * L, L)] = zero
            return 0

        lax.fori_loop(0, n_nodes // L, zr, 0)
        pltpu.sync_copy(dst_hbm.at[w], stage)
        ones = jnp.ones((L,), F32)

        def row(r, _):
            for g in range(EW // L):
                idx = stage[r, pl.ds(g * L, L)]
                plsc.addupdate_scatter(hist, [idx], ones)
            return 0

        lax.fori_loop(0, rpt, row, 0)
        pltpu.sync_copy(hist, out_hbm.at[pl.ds(w * n_nodes, n_nodes)])

    return pl.kernel(
        body,
        out_type=jax.ShapeDtypeStruct((NW * n_nodes,), F32),
        mesh=_mesh(),
        compiler_params=pltpu.CompilerParams(needs_layout_passes=False),
        scratch_types=[
            pltpu.VMEM((rpt, EW), I32),
            pltpu.VMEM((n_nodes,), F32),
        ],
    )


def _sc_msg(n_nodes, n_edges, d):
    CH = 50
    SB = 40
    ept = n_edges // NW
    rpt = ept // CH
    nsb = rpt // SB
    ZB = 8
    nblk = n_nodes // ZB
    kmax = (nblk + NS - 1) // NS

    def body(y_hbm, src_hbm, dst_hbm, out_hbm,
             sidx, didx, rows0, rows1, rows2, rows3, zbuf, acc,
             sem0, sem1, sem2, sem3):
        cid = lax.axis_index("c")
        sid = lax.axis_index("s")
        w = sid * NC + cid
        zero = jnp.zeros((L,), F32)
        dl = d // L

        def zb(i, _):
            zbuf[i // dl, pl.ds((i % dl) * L, L)] = zero
            return 0

        lax.fori_loop(0, ZB * dl, zb, 0)

        def zblk(k, _):
            b = k * NS + sid

            @pl.when(b < nblk)
            def _():
                pltpu.sync_copy(zbuf, acc.at[pl.ds(b * ZB, ZB)])
            return 0

        lax.fori_loop(0, kmax, zblk, 0)
        plsc.subcore_barrier()

        bufs = (rows0, rows1, rows2, rows3)
        sems = (sem0, sem1, sem2, sem3)

        def sblk(k, _):
            pltpu.sync_copy(src_hbm.at[w, pl.ds(k * SB, SB)], sidx)
            pltpu.sync_copy(dst_hbm.at[w, pl.ds(k * SB, SB)], didx)
            for j in range(4):
                pltpu.async_copy(y_hbm.at[sidx.at[j]], bufs[j], sems[j])

            def quad(i, _):
                for j in range(4):
                    g = i * 4 + j
                    pltpu.make_async_copy(y_hbm.at[sidx.at[g]], bufs[j],
                                          sems[j]).wait()
                    pltpu.sync_copy(bufs[j], acc.at[didx.at[g]], add=True)

                    @pl.when(g + 4 < SB)
                    def _():
                        pltpu.async_copy(y_hbm.at[sidx.at[g + 4]], bufs[j],
                                         sems[j])
                return 0

            lax.fori_loop(0, SB // 4, quad, 0)
            return 0

        lax.fori_loop(0, nsb, sblk, 0)
        plsc.subcore_barrier()

        def dblk(k, _):
            b = k * NS + sid

            @pl.when(b < nblk)
            def _():
                pltpu.sync_copy(acc.at[pl.ds(b * ZB, ZB)],
                                out_hbm.at[cid, pl.ds(b * ZB, ZB)])
            return 0

        lax.fori_loop(0, kmax, dblk, 0)

    return pl.kernel(
        body,
        out_type=jax.ShapeDtypeStruct((NC, n_nodes, d), F32),
        mesh=_mesh(),
        compiler_params=pltpu.CompilerParams(needs_layout_passes=False),
        scratch_types=[
            pltpu.VMEM((SB, CH), I32),
            pltpu.VMEM((SB, CH), I32),
            pltpu.VMEM((CH, d), F32),
            pltpu.VMEM((CH, d), F32),
            pltpu.VMEM((CH, d), F32),
            pltpu.VMEM((CH, d), F32),
            pltpu.VMEM((ZB, d), F32),
            pltpu.VMEM_SHARED((n_nodes, d), F32),
            pltpu.SemaphoreType.DMA,
            pltpu.SemaphoreType.DMA,
            pltpu.SemaphoreType.DMA,
            pltpu.SemaphoreType.DMA,
        ],
    )


def _sc_stats(n_nodes, d, n_clusters, n_edges):
    nrow = n_nodes // EW
    ept = n_edges // NW
    erpt = ept // EW
    ZB = 80
    nblk = n_clusters // ZB
    kmax = (nblk + NS - 1) // NS

    def body(h_hbm, clu_hbm, srcg, dstg,
             out_hbm, cnt_hbm, eid_hbm,
             cstage, rowbuf, hist, clu_v, sstage, dstage, zbuf, acc):
        cid = lax.axis_index("c")
        sid = lax.axis_index("s")
        w = sid * NC + cid
        zero = jnp.zeros((L,), F32)
        dl = d // L

        def zb(i, _):
            zbuf[i // dl, pl.ds((i % dl) * L, L)] = zero
            return 0

        lax.fori_loop(0, ZB * dl, zb, 0)

        def zblk(k, _):
            b = k * NS + sid

            @pl.when(b < nblk)
            def _():
                pltpu.sync_copy(zbuf, acc.at[pl.ds(b * ZB, ZB)])
            return 0

        lax.fori_loop(0, kmax, zblk, 0)

        def zh(i, _):
            hist[pl.ds(i * L, L)] = zero
            return 0

        lax.fori_loop(0, n_clusters // L, zh, 0)
        pltpu.sync_copy(clu_hbm, clu_v)
        plsc.subcore_barrier()

        ones = jnp.ones((L,), F32)
        lo = (w * nrow) // NW
        hi = ((w + 1) * nrow) // NW

        def row(r, _):
            for g in range(EW // L):
                v = clu_v[pl.ds(r * EW + g * L, L)]
                cstage[pl.ds(g * L, L)] = v
                plsc.addupdate_scatter(hist, [v], ones)
            pltpu.sync_copy(h_hbm.at[pl.ds(r * EW, EW)], rowbuf)
            pltpu.sync_copy(rowbuf, acc.at[cstage], add=True)
            return 0

        lax.fori_loop(lo, hi, row, 0)

        pltpu.sync_copy(srcg.at[w], sstage)
        pltpu.sync_copy(dstg.at[w], dstage)

        def erowf(r, _):
            for g in range(EW // L):
                sv = sstage[r, pl.ds(g * L, L)]
                dv = dstage[r, pl.ds(g * L, L)]
                cs = plsc.load_gather(clu_v, [sv])
                cd = plsc.load_gather(clu_v, [dv])
                sstage[r, pl.ds(g * L, L)] = cs * n_clusters + cd
            return 0

        lax.fori_loop(0, erpt, erowf, 0)
        pltpu.sync_copy(sstage, eid_hbm.at[w])
        pltpu.sync_copy(hist, cnt_hbm.at[pl.ds(w * n_clusters, n_clusters)])
        plsc.subcore_barrier()

        def dblk(k, _):
            b = k * NS + sid

            @pl.when(b < nblk)
            def _():
                pltpu.sync_copy(acc.at[pl.ds(b * ZB, ZB)],
                                out_hbm.at[cid, pl.ds(b * ZB, ZB)])
            return 0

        lax.fori_loop(0, kmax, dblk, 0)

    return pl.kernel(
        body,
        out_type=[
            jax.ShapeDtypeStruct((NC, n_clusters, d), F32),
            jax.ShapeDtypeStruct((NW * n_clusters,), F32),
            jax.ShapeDtypeStruct((NW, n_edges // NW // EW, EW), I32),
        ],
        mesh=_mesh(),
        compiler_params=pltpu.CompilerParams(needs_layout_passes=False),
        scratch_types=[
            pltpu.VMEM((EW,), I32),
            pltpu.VMEM((EW, d), F32),
            pltpu.VMEM((n_clusters,), F32),
            pltpu.VMEM((n_nodes,), I32),
            pltpu.VMEM((erpt, EW), I32),
            pltpu.VMEM((erpt, EW), I32),
            pltpu.VMEM((ZB, d), F32),
            pltpu.VMEM_SHARED((n_clusters, d), F32),
        ],
    )


def _sc_ahist(n_clusters, n_edges):
    bins = n_clusters * n_clusters // NW
    chunk = n_edges // NW

    def body(eidf, out_hbm, estage, hist):
        w = _worker_id()
        lo_bin = w * bins
        zero = jnp.zeros((L,), F32)
        ones = jnp.ones((L,), F32)

        def zr(i, _):
            hist[pl.ds(i * L, L)] = zero
            return 0

        lax.fori_loop(0, bins // L, zr, 0)

        def ch(ci, _):
            pltpu.sync_copy(eidf.at[pl.ds(ci * chunk, chunk)], estage)

            def grp(g, _):
                for u in range(5):
                    e = estage[pl.ds(g * (5 * L) + u * L, L)]
                    loc = e - lo_bin
                    m = loc.astype(jnp.uint32) < jnp.uint32(bins)
                    locc = jnp.clip(loc, 0, bins - 1)
                    plsc.addupdate_scatter(hist, [locc], ones, mask=m)
                return 0

            lax.fori_loop(0, chunk // (5 * L), grp, 0)
            return 0

        lax.fori_loop(0, NW, ch, 0)
        pltpu.sync_copy(hist, out_hbm.at[pl.ds(w * bins, bins)])

    return pl.kernel(
        body,
        out_type=jax.ShapeDtypeStruct((NW * bins,), F32),
        mesh=_mesh(),
        compiler_params=pltpu.CompilerParams(needs_layout_passes=False),
        scratch_types=[
            pltpu.VMEM((chunk,), I32),
            pltpu.VMEM((bins,), F32),
        ],
    )


def _tc_prep(n, d, h):
    def body(part, x, w, dinv_ref, y_ref):
        deg = lax.dot_general(part[...], jnp.ones((NW, 1), F32),
                              (((0,), (0,)), ((), ())),
                              preferred_element_type=F32) + 1.0
        dv = lax.rsqrt(deg)
        dinv_ref[...] = dv
        y_ref[...] = jnp.dot(x[...], w[...], preferred_element_type=F32) * dv

    return pl.pallas_call(
        body,
        out_shape=[jax.ShapeDtypeStruct((n, 1), F32),
                   jax.ShapeDtypeStruct((n, h), F32)],
    )


def _tc_layer2(n, h):
    def body(acc, y1, dinv, b1, w2, y2_ref):
        h1 = jnp.maximum(dinv[...] * (acc[0] + acc[1] + y1[...]) + b1[...], 0.0)
        y2_ref[...] = jnp.dot(h1, w2[...], preferred_element_type=F32) * dinv[...]

    return pl.pallas_call(
        body,
        out_shape=jax.ShapeDtypeStruct((n, h), F32),
    )


def _tc_kmeans(n, d):
    PAD = 1024
    NP = n + PAD
    BLK = 768

    def body(acc, y2, dinv, b2, batch, h_ref, clu_ref, hpad, bpad, clpad):
        hh = dinv[...] * (acc[0] + acc[1] + y2[...]) + b2[...]
        h_ref[...] = hh
        hpad[0:n, :] = hh
        hpad[n:NP, :] = jnp.zeros((PAD, d), F32)
        bv = batch[...]
        bpad[0:n, :] = bv
        bpad[n:NP, :] = jnp.full((PAD, 1), NB, I32)

        def bloop(bi, _):
            start = jnp.sum((bv < bi).astype(I32))
            n_end = jnp.sum((bv < bi + 1).astype(I32))
            ws = pl.multiple_of((start // 8) * 8, 8)
            nblk = (n_end - ws + BLK - 1) // BLK
            c = hpad[pl.ds(start, KC), :]
            iota_k = lax.broadcasted_iota(I32, (BLK, KC), 1)
            onesd = jnp.ones((1, d), F32)
            onesb = jnp.ones((BLK, 1), F32)
            for it in range(KM_IT):
                csq = lax.dot_general(onesd, c * c, (((1,), (1,)), ((), ())),
                                      preferred_element_type=F32)
                last = it == KM_IT - 1

                def blk(j, carry):
                    r0 = ws + j * BLK
                    hb = hpad[pl.ds(r0, BLK), :]
                    valid = bpad[pl.ds(r0, BLK), :] == bi
                    xsq = jnp.sum(hb * hb, axis=1, keepdims=True)
                    d2 = xsq + csq - 2.0 * lax.dot_general(
                        hb, c, (((1,), (1,)), ((), ())),
                        preferred_element_type=F32)
                    mn = jnp.min(d2, axis=1, keepdims=True)
                    ass = jnp.min(jnp.where(d2 <= mn, iota_k, KC),
                                  axis=1, keepdims=True)
                    if last:
                        old = clpad[pl.ds(r0, BLK), :]
                        clpad[pl.ds(r0, BLK), :] = jnp.where(valid, ass + bi * KC, old)
                        return carry
                    oneh = ((iota_k == ass) & valid).astype(F32)
                    cs_, cn_ = carry
                    cs_ = cs_ + lax.dot_general(oneh, hb, (((0,), (0,)), ((), ())),
                                                preferred_element_type=F32)
                    cn_ = cn_ + lax.dot_general(oneh, onesb, (((0,), (0,)), ((), ())),
                                                preferred_element_type=F32)
                    return cs_, cn_

                if last:
                    lax.fori_loop(0, nblk, blk, 0)
                else:
                    cs_, cn_ = lax.fori_loop(
                        0, nblk, blk,
                        (jnp.zeros((KC, d), F32), jnp.zeros((KC, 1), F32)))
                    c = cs_ / jnp.maximum(cn_, 1.0)
            return 0

        lax.fori_loop(0, NB, bloop, 0)
        clu_ref[...] = clpad[0:n, :]

    return pl.pallas_call(
        body,
        out_shape=[jax.ShapeDtypeStruct((n, d), F32),
                   jax.ShapeDtypeStruct((n, 1), I32)],
        compiler_params=pltpu.CompilerParams(vmem_limit_bytes=63 * 1024 * 1024),
        scratch_shapes=[
            pltpu.VMEM((NP, d), F32),
            pltpu.VMEM((NP, 1), I32),
            pltpu.VMEM((NP, 1), I32),
        ],
    )


def _tc_final(d, out_dim):
    def body(A, cxs, cntp, wpr, bpr, w1, b1, w2, b2, wm1, bm1, wm2, bm2, out_ref):
        cnt = lax.dot_general(cntp[...], jnp.ones((NW, 1), F32),
                              (((0,), (0,)), ((), ())),
                              preferred_element_type=F32)
        cx = (cxs[0] + cxs[1]) / jnp.maximum(cnt, 1.0)
        cx = jnp.dot(cx, wpr[...], preferred_element_type=F32) + bpr[...]
        rr = lax.broadcasted_iota(I32, (CC, 1), 0)
        ccol = lax.broadcasted_iota(I32, (1, CC), 1)
        ind = jnp.where((A[...] > 0.0) & (rr != ccol), 1.0, 0.0)
        degc = lax.dot_general(ind, jnp.ones((CC, 1), F32),
                               (((0,), (0,)), ((), ())),
                               preferred_element_type=F32) + 1.0
        dinvc = lax.rsqrt(degc)

        def conv(xin, w, bias):
            z = jnp.dot(xin, w[...], preferred_element_type=F32) * dinvc
            t = lax.dot_general(ind, z, (((0,), (0,)), ((), ())),
                                preferred_element_type=F32)
            return dinvc * (t + z) + bias[...]

        h1 = jnp.maximum(conv(cx, w1, b1), 0.0)
        h2 = conv(h1, w2, b2)
        gi = lax.broadcasted_iota(I32, (NB, CC), 0)
        ci = lax.broadcasted_iota(I32, (NB, CC), 1)
        P = (ci // KC == gi).astype(F32)
        pooled = jnp.dot(P, h2, preferred_element_type=F32) / float(KC)
        hm = jnp.dot(pooled, wm1[...], preferred_element_type=F32) + bm1[...]
        hm = 0.5 * hm * (1.0 + lax.erf(hm * (2.0 ** -0.5)))
        out_ref[...] = jnp.dot(hm, wm2[...], preferred_element_type=F32) + bm2[...]

    return pl.pallas_call(
        body,
        out_shape=jax.ShapeDtypeStruct((NB, out_dim), F32),
    )


def kernel(x, edge_index, batch, W_g1, b_g1, W_g2, b_g2, W_proj, b_proj,
           W_p1, b_p1, W_p2, b_p2, W_m1, b_m1, W_m2, b_m2):
    n, d = x.shape
    e = edge_index.shape[1]
    h = W_g1.shape[1]
    out_dim = W_m2.shape[1]
    ept = e // NW

    src = edge_index[0]
    dst = edge_index[1]
    src3 = src.reshape(NW, ept // EW, EW)
    dst3 = dst.reshape(NW, ept // EW, EW)

    degp = _sc_deg(n, e)(dst3).reshape(NW, n)
    dinv, y1 = _tc_prep(n, d, h)(degp, x, W_g1)
    src4 = src.reshape(NW, ept // 50, 50)
    dst4 = dst.reshape(NW, ept // 50, 50)
    acc1 = _sc_msg(n, e, h)(y1, src4, dst4)
    y2 = _tc_layer2(n, h)(acc1, y1, dinv, b_g1.reshape(1, h), W_g2)
    acc2 = _sc_msg(n, e, h)(y2, src4, dst4)
    hfeat, clu2d = _tc_kmeans(n, h)(acc2, y2, dinv, b_g2.reshape(1, h),
                                    batch.reshape(n, 1))
    clu = clu2d.reshape(n)
    cxs, cntf, eid3 = _sc_stats(n, h, CC, e)(hfeat, clu, src3, dst3)
    cntp = cntf.reshape(NW, CC)
    acnt = _sc_ahist(CC, e)(eid3.reshape(e))
    A2 = acnt.reshape(CC, CC)
    return _tc_final(h, out_dim)(
        A2, cxs, cntp, W_proj, b_proj.reshape(1, h), W_p1, b_p1.reshape(1, h),
        W_p2, b_p2.reshape(1, h), W_m1, b_m1.reshape(1, h),
        W_m2, b_m2.reshape(1, out_dim))


# speedup vs baseline: 28.7837x; 1.0377x over previous
import jax
import jax.numpy as jnp
from jax import lax
from jax.experimental import pallas as pl
from jax.experimental.pallas import tpu as pltpu
from jax.experimental.pallas import tpu_sc as plsc

F32 = jnp.float32
I32 = jnp.int32

NC, NS, L = 2, 16, 16
NW = NC * NS

NB = 16
KC = 100
CC = NB * KC
KM_IT = 5

EW = 80


def _mesh():
    return plsc.VectorSubcoreMesh(core_axis_name="c", subcore_axis_name="s")


def _worker_id():
    return lax.axis_index("s") * NC + lax.axis_index("c")


def _sc_deg(n_nodes, n_edges):
    ept = n_edges // NW
    rpt = ept // EW

    def body(dst_hbm, out_hbm, stage, hist):
        w = _worker_id()
        zero = jnp.zeros((L,), F32)

        def zr(i, _):
            hist[pl.ds(i * L, L)] = zero
            return 0

        lax.fori_loop(0, n_nodes // L, zr, 0)
        pltpu.sync_copy(dst_hbm.at[w], stage)
        ones = jnp.ones((L,), F32)

        def row(r, _):
            for g in range(EW // L):
                idx = stage[r, pl.ds(g * L, L)]
                plsc.addupdate_scatter(hist, [idx], ones)
            return 0

        lax.fori_loop(0, rpt, row, 0)
        pltpu.sync_copy(hist, out_hbm.at[pl.ds(w * n_nodes, n_nodes)])

    return pl.kernel(
        body,
        out_type=jax.ShapeDtypeStruct((NW * n_nodes,), F32),
        mesh=_mesh(),
        compiler_params=pltpu.CompilerParams(needs_layout_passes=False),
        scratch_types=[
            pltpu.VMEM((rpt, EW), I32),
            pltpu.VMEM((n_nodes,), F32),
        ],
    )


def _sc_msg(n_nodes, n_edges, d):
    CH = 50
    SB = 40
    ept = n_edges // NW
    rpt = ept // CH
    nsb = rpt // SB
    ZB = 8
    nblk = n_nodes // ZB
    kmax = (nblk + NS - 1) // NS

    def body(y_hbm, src_hbm, dst_hbm, out_hbm,
             sidx, didx, rows0, rows1, rows2, rows3, zbuf, acc,
             sem0, sem1, sem2, sem3):
        cid = lax.axis_index("c")
        sid = lax.axis_index("s")
        w = sid * NC + cid
        zero = jnp.zeros((L,), F32)
        dl = d // L

        def zb(i, _):
            zbuf[i // dl, pl.ds((i % dl) * L, L)] = zero
            return 0

        lax.fori_loop(0, ZB * dl, zb, 0)

        def zblk(k, _):
            b = k * NS + sid

            @pl.when(b < nblk)
            def _():
                pltpu.sync_copy(zbuf, acc.at[pl.ds(b * ZB, ZB)])
            return 0

        lax.fori_loop(0, kmax, zblk, 0)
        plsc.subcore_barrier()

        bufs = (rows0, rows1, rows2, rows3)
        sems = (sem0, sem1, sem2, sem3)

        def sblk(k, _):
            pltpu.sync_copy(src_hbm.at[w, pl.ds(k * SB, SB)], sidx)
            pltpu.sync_copy(dst_hbm.at[w, pl.ds(k * SB, SB)], didx)
            for j in range(4):
                pltpu.async_copy(y_hbm.at[sidx.at[j]], bufs[j], sems[j])

            def quad(i, _):
                for j in range(4):
                    g = i * 4 + j
                    pltpu.make_async_copy(y_hbm.at[sidx.at[g]], bufs[j],
                                          sems[j]).wait()
                    pltpu.sync_copy(bufs[j], acc.at[didx.at[g]], add=True)

                    @pl.when(g + 4 < SB)
                    def _():
                        pltpu.async_copy(y_hbm.at[sidx.at[g + 4]], bufs[j],
                                         sems[j])
                return 0

            lax.fori_loop(0, SB // 4, quad, 0)
            return 0

        lax.fori_loop(0, nsb, sblk, 0)
        plsc.subcore_barrier()

        def dblk(k, _):
            b = k * NS + sid

            @pl.when(b < nblk)
            def _():
                pltpu.sync_copy(acc.at[pl.ds(b * ZB, ZB)],
                                out_hbm.at[cid, pl.ds(b * ZB, ZB)])
            return 0

        lax.fori_loop(0, kmax, dblk, 0)

    return pl.kernel(
        body,
        out_type=jax.ShapeDtypeStruct((NC, n_nodes, d), F32),
        mesh=_mesh(),
        compiler_params=pltpu.CompilerParams(needs_layout_passes=False),
        scratch_types=[
            pltpu.VMEM((SB, CH), I32),
            pltpu.VMEM((SB, CH), I32),
            pltpu.VMEM((CH, d), F32),
            pltpu.VMEM((CH, d), F32),
            pltpu.VMEM((CH, d), F32),
            pltpu.VMEM((CH, d), F32),
            pltpu.VMEM((ZB, d), F32),
            pltpu.VMEM_SHARED((n_nodes, d), F32),
            pltpu.SemaphoreType.DMA,
            pltpu.SemaphoreType.DMA,
            pltpu.SemaphoreType.DMA,
            pltpu.SemaphoreType.DMA,
        ],
    )


def _sc_stats(n_nodes, d, n_clusters, n_edges):
    nrow = n_nodes // EW
    ept = n_edges // NW
    erpt = ept // EW
    ZB = 80
    nblk = n_clusters // ZB
    kmax = (nblk + NS - 1) // NS

    def body(h_hbm, clu_hbm, srcg, dstg,
             out_hbm, cnt_hbm, eid_hbm,
             cstage, rowbuf, hist, clu_v, sstage, dstage, zbuf, acc):
        cid = lax.axis_index("c")
        sid = lax.axis_index("s")
        w = sid * NC + cid
        zero = jnp.zeros((L,), F32)
        dl = d // L

        def zb(i, _):
            zbuf[i // dl, pl.ds((i % dl) * L, L)] = zero
            return 0

        lax.fori_loop(0, ZB * dl, zb, 0)

        def zblk(k, _):
            b = k * NS + sid

            @pl.when(b < nblk)
            def _():
                pltpu.sync_copy(zbuf, acc.at[pl.ds(b * ZB, ZB)])
            return 0

        lax.fori_loop(0, kmax, zblk, 0)

        def zh(i, _):
            hist[pl.ds(i * L, L)] = zero
            return 0

        lax.fori_loop(0, n_clusters // L, zh, 0)
        pltpu.sync_copy(clu_hbm, clu_v)
        plsc.subcore_barrier()

        ones = jnp.ones((L,), F32)
        lo = (w * nrow) // NW
        hi = ((w + 1) * nrow) // NW

        def row(r, _):
            for g in range(EW // L):
                v = clu_v[pl.ds(r * EW + g * L, L)]
                cstage[pl.ds(g * L, L)] = v
                plsc.addupdate_scatter(hist, [v], ones)
            pltpu.sync_copy(h_hbm.at[pl.ds(r * EW, EW)], rowbuf)
            pltpu.sync_copy(rowbuf, acc.at[cstage], add=True)
            return 0

        lax.fori_loop(lo, hi, row, 0)

        pltpu.sync_copy(srcg.at[w], sstage)
        pltpu.sync_copy(dstg.at[w], dstage)

        def erowf(r, _):
            for g in range(EW // L):
                sv = sstage[r, pl.ds(g * L, L)]
                dv = dstage[r, pl.ds(g * L, L)]
                cs = plsc.load_gather(clu_v, [sv])
                cd = plsc.load_gather(clu_v, [dv])
                sstage[r, pl.ds(g * L, L)] = cs * n_clusters + cd
            return 0

        lax.fori_loop(0, erpt, erowf, 0)
        pltpu.sync_copy(sstage, eid_hbm.at[w])
        pltpu.sync_copy(hist, cnt_hbm.at[pl.ds(w * n_clusters, n_clusters)])
        plsc.subcore_barrier()

        def dblk(k, _):
            b = k * NS + sid

            @pl.when(b < nblk)
            def _():
                pltpu.sync_copy(acc.at[pl.ds(b * ZB, ZB)],
                                out_hbm.at[cid, pl.ds(b * ZB, ZB)])
            return 0

        lax.fori_loop(0, kmax, dblk, 0)

    return pl.kernel(
        body,
        out_type=[
            jax.ShapeDtypeStruct((NC, n_clusters, d), F32),
            jax.ShapeDtypeStruct((NW * n_clusters,), F32),
            jax.ShapeDtypeStruct((NW, n_edges // NW // EW, EW), I32),
        ],
        mesh=_mesh(),
        compiler_params=pltpu.CompilerParams(needs_layout_passes=False),
        scratch_types=[
            pltpu.VMEM((EW,), I32),
            pltpu.VMEM((EW, d), F32),
            pltpu.VMEM((n_clusters,), F32),
            pltpu.VMEM((n_nodes,), I32),
            pltpu.VMEM((erpt, EW), I32),
            pltpu.VMEM((erpt, EW), I32),
            pltpu.VMEM((ZB, d), F32),
            pltpu.VMEM_SHARED((n_clusters, d), F32),
        ],
    )


def _sc_ahist(n_clusters, n_edges):
    bins = n_clusters * n_clusters // NW
    chunk = n_edges // NW

    def body(eidf, out_hbm, estage, hist):
        w = _worker_id()
        lo_bin = w * bins
        zero = jnp.zeros((L,), F32)
        ones = jnp.ones((L,), F32)

        def zr(i, _):
            hist[pl.ds(i * L, L)] = zero
            return 0

        lax.fori_loop(0, bins // L, zr, 0)

        def ch(ci, _):
            pltpu.sync_copy(eidf.at[pl.ds(ci * chunk, chunk)], estage)

            def grp(g, _):
                for u in range(5):
                    e = estage[pl.ds(g * (5 * L) + u * L, L)]
                    loc = e - lo_bin
                    m = loc.astype(jnp.uint32) < jnp.uint32(bins)
                    locc = jnp.clip(loc, 0, bins - 1)
                    plsc.addupdate_scatter(hist, [locc], ones, mask=m)
                return 0

            lax.fori_loop(0, chunk // (5 * L), grp, 0)
            return 0

        lax.fori_loop(0, NW, ch, 0)
        pltpu.sync_copy(hist, out_hbm.at[pl.ds(w * bins, bins)])

    return pl.kernel(
        body,
        out_type=jax.ShapeDtypeStruct((NW * bins,), F32),
        mesh=_mesh(),
        compiler_params=pltpu.CompilerParams(needs_layout_passes=False),
        scratch_types=[
            pltpu.VMEM((chunk,), I32),
            pltpu.VMEM((bins,), F32),
        ],
    )


def _tc_prep(n, d, h):
    def body(part, x, w, dinv_ref, y_ref):
        deg = lax.dot_general(part[...], jnp.ones((NW, 1), F32),
                              (((0,), (0,)), ((), ())),
                              preferred_element_type=F32) + 1.0
        dv = lax.rsqrt(deg)
        dinv_ref[...] = dv
        y_ref[...] = jnp.dot(x[...], w[...], preferred_element_type=F32) * dv

    return pl.pallas_call(
        body,
        out_shape=[jax.ShapeDtypeStruct((n, 1), F32),
                   jax.ShapeDtypeStruct((n, h), F32)],
    )


def _tc_layer2(n, h):
    def body(acc, y1, dinv, b1, w2, y2_ref):
        h1 = jnp.maximum(dinv[...] * (acc[0] + acc[1] + y1[...]) + b1[...], 0.0)
        y2_ref[...] = jnp.dot(h1, w2[...], preferred_element_type=F32) * dinv[...]

    return pl.pallas_call(
        body,
        out_shape=jax.ShapeDtypeStruct((n, h), F32),
    )


def _tc_kmeans(n, d):
    PAD = 1024
    NP = n + PAD
    BLK = 768

    def body(acc, y2, dinv, b2, batch, h_ref, clu_ref, hpad, bpad, clpad):
        hh = dinv[...] * (acc[0] + acc[1] + y2[...]) + b2[...]
        h_ref[...] = hh
        hpad[0:n, :] = hh
        hpad[n:NP, :] = jnp.zeros((PAD, d), F32)
        bv = batch[...]
        bpad[0:n, :] = bv
        bpad[n:NP, :] = jnp.full((PAD, 1), NB, I32)

        brow = lax.broadcasted_iota(I32, (1, NB + 1), 1)
        srow = jnp.sum((bv < brow).astype(I32), axis=0, keepdims=True)

        def bloop(bi, _):
            start = jnp.sum(jnp.where(brow == bi, srow, 0))
            n_end = jnp.sum(jnp.where(brow == bi + 1, srow, 0))
            ws = pl.multiple_of((start // 8) * 8, 8)
            nblk = (n_end - ws + BLK - 1) // BLK
            c = hpad[pl.ds(start, KC), :]
            iota_k = lax.broadcasted_iota(I32, (BLK, KC), 1)
            onesd = jnp.ones((1, d), F32)
            onesb = jnp.ones((BLK, 1), F32)
            for it in range(KM_IT):
                csq = lax.dot_general(onesd, c * c, (((1,), (1,)), ((), ())),
                                      preferred_element_type=F32)
                last = it == KM_IT - 1

                def blk(j, carry):
                    r0 = ws + j * BLK
                    hb = hpad[pl.ds(r0, BLK), :]
                    valid = bpad[pl.ds(r0, BLK), :] == bi
                    xsq = jnp.sum(hb * hb, axis=1, keepdims=True)
                    d2 = xsq + csq - 2.0 * lax.dot_general(
                        hb, c, (((1,), (1,)), ((), ())),
                        preferred_element_type=F32)
                    mn = jnp.min(d2, axis=1, keepdims=True)
                    ass = jnp.min(jnp.where(d2 <= mn, iota_k, KC),
                                  axis=1, keepdims=True)
                    if last:
                        old = clpad[pl.ds(r0, BLK), :]
                        clpad[pl.ds(r0, BLK), :] = jnp.where(valid, ass + bi * KC, old)
                        return carry
                    oneh = ((iota_k == ass) & valid).astype(F32)
                    cs_, cn_ = carry
                    cs_ = cs_ + lax.dot_general(oneh, hb, (((0,), (0,)), ((), ())),
                                                preferred_element_type=F32)
                    cn_ = cn_ + lax.dot_general(oneh, onesb, (((0,), (0,)), ((), ())),
                                                preferred_element_type=F32)
                    return cs_, cn_

                if last:
                    lax.fori_loop(0, nblk, blk, 0)
                else:
                    cs_, cn_ = lax.fori_loop(
                        0, nblk, blk,
                        (jnp.zeros((KC, d), F32), jnp.zeros((KC, 1), F32)))
                    c = cs_ / jnp.maximum(cn_, 1.0)
            return 0

        lax.fori_loop(0, NB, bloop, 0)
        clu_ref[...] = clpad[0:n, :]

    return pl.pallas_call(
        body,
        out_shape=[jax.ShapeDtypeStruct((n, d), F32),
                   jax.ShapeDtypeStruct((n, 1), I32)],
        compiler_params=pltpu.CompilerParams(vmem_limit_bytes=63 * 1024 * 1024),
        scratch_shapes=[
            pltpu.VMEM((NP, d), F32),
            pltpu.VMEM((NP, 1), I32),
            pltpu.VMEM((NP, 1), I32),
        ],
    )


def _tc_final(d, out_dim):
    def body(A, cxs, cntp, wpr, bpr, w1, b1, w2, b2, wm1, bm1, wm2, bm2, out_ref):
        cnt = lax.dot_general(cntp[...], jnp.ones((NW, 1), F32),
                              (((0,), (0,)), ((), ())),
                              preferred_element_type=F32)
        cx = (cxs[0] + cxs[1]) / jnp.maximum(cnt, 1.0)
        cx = jnp.dot(cx, wpr[...], preferred_element_type=F32) + bpr[...]
        rr = lax.broadcasted_iota(I32, (CC, 1), 0)
        ccol = lax.broadcasted_iota(I32, (1, CC), 1)
        ind = jnp.where((A[...] > 0.0) & (rr != ccol), 1.0, 0.0)
        degc = lax.dot_general(ind, jnp.ones((CC, 1), F32),
                               (((0,), (0,)), ((), ())),
                               preferred_element_type=F32) + 1.0
        dinvc = lax.rsqrt(degc)

        def conv(xin, w, bias):
            z = jnp.dot(xin, w[...], preferred_element_type=F32) * dinvc
            t = lax.dot_general(ind, z, (((0,), (0,)), ((), ())),
                                preferred_element_type=F32)
            return dinvc * (t + z) + bias[...]

        h1 = jnp.maximum(conv(cx, w1, b1), 0.0)
        h2 = conv(h1, w2, b2)
        gi = lax.broadcasted_iota(I32, (NB, CC), 0)
        ci = lax.broadcasted_iota(I32, (NB, CC), 1)
        P = (ci // KC == gi).astype(F32)
        pooled = jnp.dot(P, h2, preferred_element_type=F32) / float(KC)
        hm = jnp.dot(pooled, wm1[...], preferred_element_type=F32) + bm1[...]
        hm = 0.5 * hm * (1.0 + lax.erf(hm * (2.0 ** -0.5)))
        out_ref[...] = jnp.dot(hm, wm2[...], preferred_element_type=F32) + bm2[...]

    return pl.pallas_call(
        body,
        out_shape=jax.ShapeDtypeStruct((NB, out_dim), F32),
    )


def kernel(x, edge_index, batch, W_g1, b_g1, W_g2, b_g2, W_proj, b_proj,
           W_p1, b_p1, W_p2, b_p2, W_m1, b_m1, W_m2, b_m2):
    n, d = x.shape
    e = edge_index.shape[1]
    h = W_g1.shape[1]
    out_dim = W_m2.shape[1]
    ept = e // NW

    src = edge_index[0]
    dst = edge_index[1]
    src3 = src.reshape(NW, ept // EW, EW)
    dst3 = dst.reshape(NW, ept // EW, EW)

    degp = _sc_deg(n, e)(dst3).reshape(NW, n)
    dinv, y1 = _tc_prep(n, d, h)(degp, x, W_g1)
    src4 = src.reshape(NW, ept // 50, 50)
    dst4 = dst.reshape(NW, ept // 50, 50)
    acc1 = _sc_msg(n, e, h)(y1, src4, dst4)
    y2 = _tc_layer2(n, h)(acc1, y1, dinv, b_g1.reshape(1, h), W_g2)
    acc2 = _sc_msg(n, e, h)(y2, src4, dst4)
    hfeat, clu2d = _tc_kmeans(n, h)(acc2, y2, dinv, b_g2.reshape(1, h),
                                    batch.reshape(n, 1))
    clu = clu2d.reshape(n)
    cxs, cntf, eid3 = _sc_stats(n, h, CC, e)(hfeat, clu, src3, dst3)
    cntp = cntf.reshape(NW, CC)
    acnt = _sc_ahist(CC, e)(eid3.reshape(e))
    A2 = acnt.reshape(CC, CC)
    return _tc_final(h, out_dim)(
        A2, cxs, cntp, W_proj, b_proj.reshape(1, h), W_p1, b_p1.reshape(1, h),
        W_p2, b_p2.reshape(1, h), W_m1, b_m1.reshape(1, h),
        W_m2, b_m2.reshape(1, out_dim))


# speedup vs baseline: 30.1498x; 1.0475x over previous
import jax
import jax.numpy as jnp
from jax import lax
from jax.experimental import pallas as pl
from jax.experimental.pallas import tpu as pltpu
from jax.experimental.pallas import tpu_sc as plsc

F32 = jnp.float32
I32 = jnp.int32

NC, NS, L = 2, 16, 16
NW = NC * NS

NB = 16
KC = 100
CC = NB * KC
KM_IT = 5

EW = 80


def _mesh():
    return plsc.VectorSubcoreMesh(core_axis_name="c", subcore_axis_name="s")


def _worker_id():
    return lax.axis_index("s") * NC + lax.axis_index("c")


def _sc_deg(n_nodes, n_edges):
    ept = n_edges // NW
    rpt = ept // EW

    def body(dst_hbm, out_hbm, stage, hist):
        w = _worker_id()
        zero = jnp.zeros((L,), F32)

        def zr(i, _):
            hist[pl.ds(i * L, L)] = zero
            return 0

        lax.fori_loop(0, n_nodes // L, zr, 0)
        pltpu.sync_copy(dst_hbm.at[w], stage)
        ones = jnp.ones((L,), F32)

        def row(r, _):
            for g in range(EW // L):
                idx = stage[r, pl.ds(g * L, L)]
                plsc.addupdate_scatter(hist, [idx], ones)
            return 0

        lax.fori_loop(0, rpt, row, 0)
        pltpu.sync_copy(hist, out_hbm.at[pl.ds(w * n_nodes, n_nodes)])

    return pl.kernel(
        body,
        out_type=jax.ShapeDtypeStruct((NW * n_nodes,), F32),
        mesh=_mesh(),
        compiler_params=pltpu.CompilerParams(needs_layout_passes=False),
        scratch_types=[
            pltpu.VMEM((rpt, EW), I32),
            pltpu.VMEM((n_nodes,), F32),
        ],
    )


def _sc_msg(n_nodes, n_edges, d):
    CH = 50
    SB = 40
    ept = n_edges // NW
    rpt = ept // CH
    nsb = rpt // SB
    ZB = 8
    nblk = n_nodes // ZB
    kmax = (nblk + NS - 1) // NS

    def body(y_hbm, src_hbm, dst_hbm, out_hbm,
             sidx, didx, rows0, rows1, rows2, rows3, zbuf, acc,
             sem0, sem1, sem2, sem3):
        cid = lax.axis_index("c")
        sid = lax.axis_index("s")
        w = sid * NC + cid
        zero = jnp.zeros((L,), F32)
        dl = d // L

        def zb(i, _):
            zbuf[i // dl, pl.ds((i % dl) * L, L)] = zero
            return 0

        lax.fori_loop(0, ZB * dl, zb, 0)

        def zblk(k, _):
            b = k * NS + sid

            @pl.when(b < nblk)
            def _():
                pltpu.sync_copy(zbuf, acc.at[pl.ds(b * ZB, ZB)])
            return 0

        lax.fori_loop(0, kmax, zblk, 0)
        plsc.subcore_barrier()

        bufs = (rows0, rows1, rows2, rows3)
        sems = (sem0, sem1, sem2, sem3)

        def sblk(k, _):
            pltpu.sync_copy(src_hbm.at[w, pl.ds(k * SB, SB)], sidx)
            pltpu.sync_copy(dst_hbm.at[w, pl.ds(k * SB, SB)], didx)
            for j in range(4):
                pltpu.async_copy(y_hbm.at[sidx.at[j]], bufs[j], sems[j])

            def quad(i, _):
                for j in range(4):
                    g = i * 4 + j
                    pltpu.make_async_copy(y_hbm.at[sidx.at[g]], bufs[j],
                                          sems[j]).wait()
                    pltpu.sync_copy(bufs[j], acc.at[didx.at[g]], add=True)

                    @pl.when(g + 4 < SB)
                    def _():
                        pltpu.async_copy(y_hbm.at[sidx.at[g + 4]], bufs[j],
                                         sems[j])
                return 0

            lax.fori_loop(0, SB // 4, quad, 0)
            return 0

        lax.fori_loop(0, nsb, sblk, 0)
        plsc.subcore_barrier()

        def dblk(k, _):
            b = k * NS + sid

            @pl.when(b < nblk)
            def _():
                pltpu.sync_copy(acc.at[pl.ds(b * ZB, ZB)],
                                out_hbm.at[cid, pl.ds(b * ZB, ZB)])
            return 0

        lax.fori_loop(0, kmax, dblk, 0)

    return pl.kernel(
        body,
        out_type=jax.ShapeDtypeStruct((NC, n_nodes, d), F32),
        mesh=_mesh(),
        compiler_params=pltpu.CompilerParams(needs_layout_passes=False),
        scratch_types=[
            pltpu.VMEM((SB, CH), I32),
            pltpu.VMEM((SB, CH), I32),
            pltpu.VMEM((CH, d), F32),
            pltpu.VMEM((CH, d), F32),
            pltpu.VMEM((CH, d), F32),
            pltpu.VMEM((CH, d), F32),
            pltpu.VMEM((ZB, d), F32),
            pltpu.VMEM_SHARED((n_nodes, d), F32),
            pltpu.SemaphoreType.DMA,
            pltpu.SemaphoreType.DMA,
            pltpu.SemaphoreType.DMA,
            pltpu.SemaphoreType.DMA,
        ],
    )


def _sc_stats(n_nodes, d, n_clusters, n_edges):
    nrow = n_nodes // EW
    ept = n_edges // NW
    erpt = ept // EW
    ZB = 80
    nblk = n_clusters // ZB
    kmax = (nblk + NS - 1) // NS

    def body(h_hbm, clu_hbm, srcg, dstg,
             out_hbm, cnt_hbm, eid_hbm,
             cstage, rowbuf, hist, clu_v, sstage, dstage, zbuf, acc):
        cid = lax.axis_index("c")
        sid = lax.axis_index("s")
        w = sid * NC + cid
        zero = jnp.zeros((L,), F32)
        dl = d // L

        def zb(i, _):
            zbuf[i // dl, pl.ds((i % dl) * L, L)] = zero
            return 0

        lax.fori_loop(0, ZB * dl, zb, 0)

        def zblk(k, _):
            b = k * NS + sid

            @pl.when(b < nblk)
            def _():
                pltpu.sync_copy(zbuf, acc.at[pl.ds(b * ZB, ZB)])
            return 0

        lax.fori_loop(0, kmax, zblk, 0)

        def zh(i, _):
            hist[pl.ds(i * L, L)] = zero
            return 0

        lax.fori_loop(0, n_clusters // L, zh, 0)
        pltpu.sync_copy(clu_hbm, clu_v)
        plsc.subcore_barrier()

        ones = jnp.ones((L,), F32)
        lo = (w * nrow) // NW
        hi = ((w + 1) * nrow) // NW

        def row(r, _):
            for g in range(EW // L):
                v = clu_v[pl.ds(r * EW + g * L, L)]
                cstage[pl.ds(g * L, L)] = v
                plsc.addupdate_scatter(hist, [v], ones)
            pltpu.sync_copy(h_hbm.at[pl.ds(r * EW, EW)], rowbuf)
            pltpu.sync_copy(rowbuf, acc.at[cstage], add=True)
            return 0

        lax.fori_loop(lo, hi, row, 0)

        pltpu.sync_copy(srcg.at[w], sstage)
        pltpu.sync_copy(dstg.at[w], dstage)

        def erowf(r, _):
            for g in range(EW // L):
                sv = sstage[r, pl.ds(g * L, L)]
                dv = dstage[r, pl.ds(g * L, L)]
                cs = plsc.load_gather(clu_v, [sv])
                cd = plsc.load_gather(clu_v, [dv])
                sstage[r, pl.ds(g * L, L)] = cs * n_clusters + cd
            return 0

        lax.fori_loop(0, erpt, erowf, 0)
        pltpu.sync_copy(sstage, eid_hbm.at[w])
        pltpu.sync_copy(hist, cnt_hbm.at[pl.ds(w * n_clusters, n_clusters)])
        plsc.subcore_barrier()

        def dblk(k, _):
            b = k * NS + sid

            @pl.when(b < nblk)
            def _():
                pltpu.sync_copy(acc.at[pl.ds(b * ZB, ZB)],
                                out_hbm.at[cid, pl.ds(b * ZB, ZB)])
            return 0

        lax.fori_loop(0, kmax, dblk, 0)

    return pl.kernel(
        body,
        out_type=[
            jax.ShapeDtypeStruct((NC, n_clusters, d), F32),
            jax.ShapeDtypeStruct((NW * n_clusters,), F32),
            jax.ShapeDtypeStruct((NW, n_edges // NW // EW, EW), I32),
        ],
        mesh=_mesh(),
        compiler_params=pltpu.CompilerParams(needs_layout_passes=False),
        scratch_types=[
            pltpu.VMEM((EW,), I32),
            pltpu.VMEM((EW, d), F32),
            pltpu.VMEM((n_clusters,), F32),
            pltpu.VMEM((n_nodes,), I32),
            pltpu.VMEM((erpt, EW), I32),
            pltpu.VMEM((erpt, EW), I32),
            pltpu.VMEM((ZB, d), F32),
            pltpu.VMEM_SHARED((n_clusters, d), F32),
        ],
    )


def _sc_ahist(n_clusters, n_edges):
    bins = n_clusters * n_clusters // NW
    chunk = n_edges // NW

    def body(eidf, out_hbm, estageA, estageB, hist, semA, semB):
        w = _worker_id()
        lo_bin = w * bins
        zero = jnp.zeros((L,), F32)
        ones = jnp.ones((L,), F32)

        def zr(i, _):
            hist[pl.ds(i * L, L)] = zero
            return 0

        lax.fori_loop(0, bins // L, zr, 0)
        stages = (estageA, estageB)
        sems = (semA, semB)
        pltpu.async_copy(eidf.at[pl.ds(0, chunk)], estageA, semA)

        def ch2(cp, _):
            for b in range(2):
                ci = cp * 2 + b
                stage = stages[b]
                pltpu.make_async_copy(eidf.at[pl.ds(0, chunk)], stage,
                                      sems[b]).wait()

                @pl.when(ci + 1 < NW)
                def _():
                    pltpu.async_copy(eidf.at[pl.ds((ci + 1) * chunk, chunk)],
                                     stages[1 - b], sems[1 - b])

                def grp(g, _):
                    for u in range(5):
                        e = stage[pl.ds(g * (5 * L) + u * L, L)]
                        loc = e - lo_bin
                        m = loc.astype(jnp.uint32) < jnp.uint32(bins)
                        locc = jnp.clip(loc, 0, bins - 1)
                        plsc.addupdate_scatter(hist, [locc], ones, mask=m)
                    return 0

                lax.fori_loop(0, chunk // (5 * L), grp, 0)
            return 0

        lax.fori_loop(0, NW // 2, ch2, 0)
        pltpu.sync_copy(hist, out_hbm.at[pl.ds(w * bins, bins)])

    return pl.kernel(
        body,
        out_type=jax.ShapeDtypeStruct((NW * bins,), F32),
        mesh=_mesh(),
        compiler_params=pltpu.CompilerParams(needs_layout_passes=False),
        scratch_types=[
            pltpu.VMEM((chunk,), I32),
            pltpu.VMEM((chunk,), I32),
            pltpu.VMEM((bins,), F32),
            pltpu.SemaphoreType.DMA,
            pltpu.SemaphoreType.DMA,
        ],
    )


def _tc_prep(n, d, h):
    def body(part, x, w, dinv_ref, y_ref):
        deg = lax.dot_general(part[...], jnp.ones((NW, 1), F32),
                              (((0,), (0,)), ((), ())),
                              preferred_element_type=F32) + 1.0
        dv = lax.rsqrt(deg)
        dinv_ref[...] = dv
        y_ref[...] = jnp.dot(x[...], w[...], preferred_element_type=F32) * dv

    return pl.pallas_call(
        body,
        out_shape=[jax.ShapeDtypeStruct((n, 1), F32),
                   jax.ShapeDtypeStruct((n, h), F32)],
    )


def _tc_layer2(n, h):
    def body(acc, y1, dinv, b1, w2, y2_ref):
        h1 = jnp.maximum(dinv[...] * (acc[0] + acc[1] + y1[...]) + b1[...], 0.0)
        y2_ref[...] = jnp.dot(h1, w2[...], preferred_element_type=F32) * dinv[...]

    return pl.pallas_call(
        body,
        out_shape=jax.ShapeDtypeStruct((n, h), F32),
    )


def _tc_kmeans(n, d):
    PAD = 1024
    NP = n + PAD
    BLK = 768

    def body(acc, y2, dinv, b2, batch, h_ref, clu_ref, hpad, bpad, clpad):
        hh = dinv[...] * (acc[0] + acc[1] + y2[...]) + b2[...]
        h_ref[...] = hh
        hpad[0:n, :] = hh
        hpad[n:NP, :] = jnp.zeros((PAD, d), F32)
        bv = batch[...]
        bpad[0:n, :] = bv
        bpad[n:NP, :] = jnp.full((PAD, 1), NB, I32)

        brow = lax.broadcasted_iota(I32, (1, NB + 1), 1)
        srow = jnp.sum((bv < brow).astype(I32), axis=0, keepdims=True)

        def bloop(bi, _):
            start = jnp.sum(jnp.where(brow == bi, srow, 0))
            n_end = jnp.sum(jnp.where(brow == bi + 1, srow, 0))
            ws = pl.multiple_of((start // 8) * 8, 8)
            nblk = (n_end - ws + BLK - 1) // BLK
            c = hpad[pl.ds(start, KC), :]
            iota_k = lax.broadcasted_iota(I32, (BLK, KC), 1)
            onesd = jnp.ones((1, d), F32)
            onesb = jnp.ones((BLK, 1), F32)
            for it in range(KM_IT):
                csq = lax.dot_general(onesd, c * c, (((1,), (1,)), ((), ())),
                                      preferred_element_type=F32)
                last = it == KM_IT - 1

                def blk(j, carry):
                    r0 = ws + j * BLK
                    hb = hpad[pl.ds(r0, BLK), :]
                    valid = bpad[pl.ds(r0, BLK), :] == bi
                    xsq = jnp.sum(hb * hb, axis=1, keepdims=True)
                    d2 = xsq + csq - 2.0 * lax.dot_general(
                        hb, c, (((1,), (1,)), ((), ())),
                        preferred_element_type=F32)
                    mn = jnp.min(d2, axis=1, keepdims=True)
                    ass = jnp.min(jnp.where(d2 <= mn, iota_k, KC),
                                  axis=1, keepdims=True)
                    if last:
                        old = clpad[pl.ds(r0, BLK), :]
                        clpad[pl.ds(r0, BLK), :] = jnp.where(valid, ass + bi * KC, old)
                        return carry
                    oneh = ((iota_k == ass) & valid).astype(F32)
                    cs_, cn_ = carry
                    cs_ = cs_ + lax.dot_general(oneh, hb, (((0,), (0,)), ((), ())),
                                                preferred_element_type=F32)
                    cn_ = cn_ + lax.dot_general(oneh, onesb, (((0,), (0,)), ((), ())),
                                                preferred_element_type=F32)
                    return cs_, cn_

                if last:
                    lax.fori_loop(0, nblk, blk, 0)
                else:
                    cs_, cn_ = lax.fori_loop(
                        0, nblk, blk,
                        (jnp.zeros((KC, d), F32), jnp.zeros((KC, 1), F32)))
                    c = cs_ / jnp.maximum(cn_, 1.0)
            return 0

        lax.fori_loop(0, NB, bloop, 0)
        clu_ref[...] = clpad[0:n, :]

    return pl.pallas_call(
        body,
        out_shape=[jax.ShapeDtypeStruct((n, d), F32),
                   jax.ShapeDtypeStruct((n, 1), I32)],
        compiler_params=pltpu.CompilerParams(vmem_limit_bytes=63 * 1024 * 1024),
        scratch_shapes=[
            pltpu.VMEM((NP, d), F32),
            pltpu.VMEM((NP, 1), I32),
            pltpu.VMEM((NP, 1), I32),
        ],
    )


def _tc_final(d, out_dim):
    def body(A, cxs, cntp, wpr, bpr, w1, b1, w2, b2, wm1, bm1, wm2, bm2, out_ref):
        cnt = lax.dot_general(cntp[...], jnp.ones((NW, 1), F32),
                              (((0,), (0,)), ((), ())),
                              preferred_element_type=F32)
        cx = (cxs[0] + cxs[1]) / jnp.maximum(cnt, 1.0)
        cx = jnp.dot(cx, wpr[...], preferred_element_type=F32) + bpr[...]
        rr = lax.broadcasted_iota(I32, (CC, 1), 0)
        ccol = lax.broadcasted_iota(I32, (1, CC), 1)
        ind = jnp.where((A[...] > 0.0) & (rr != ccol), 1.0, 0.0)
        degc = lax.dot_general(ind, jnp.ones((CC, 1), F32),
                               (((0,), (0,)), ((), ())),
                               preferred_element_type=F32) + 1.0
        dinvc = lax.rsqrt(degc)

        def conv(xin, w, bias):
            z = jnp.dot(xin, w[...], preferred_element_type=F32) * dinvc
            t = lax.dot_general(ind, z, (((0,), (0,)), ((), ())),
                                preferred_element_type=F32)
            return dinvc * (t + z) + bias[...]

        h1 = jnp.maximum(conv(cx, w1, b1), 0.0)
        h2 = conv(h1, w2, b2)
        gi = lax.broadcasted_iota(I32, (NB, CC), 0)
        ci = lax.broadcasted_iota(I32, (NB, CC), 1)
        P = (ci // KC == gi).astype(F32)
        pooled = jnp.dot(P, h2, preferred_element_type=F32) / float(KC)
        hm = jnp.dot(pooled, wm1[...], preferred_element_type=F32) + bm1[...]
        hm = 0.5 * hm * (1.0 + lax.erf(hm * (2.0 ** -0.5)))
        out_ref[...] = jnp.dot(hm, wm2[...], preferred_element_type=F32) + bm2[...]

    return pl.pallas_call(
        body,
        out_shape=jax.ShapeDtypeStruct((NB, out_dim), F32),
    )


def kernel(x, edge_index, batch, W_g1, b_g1, W_g2, b_g2, W_proj, b_proj,
           W_p1, b_p1, W_p2, b_p2, W_m1, b_m1, W_m2, b_m2):
    n, d = x.shape
    e = edge_index.shape[1]
    h = W_g1.shape[1]
    out_dim = W_m2.shape[1]
    ept = e // NW

    src = edge_index[0]
    dst = edge_index[1]
    src3 = src.reshape(NW, ept // EW, EW)
    dst3 = dst.reshape(NW, ept // EW, EW)

    degp = _sc_deg(n, e)(dst3).reshape(NW, n)
    dinv, y1 = _tc_prep(n, d, h)(degp, x, W_g1)
    src4 = src.reshape(NW, ept // 50, 50)
    dst4 = dst.reshape(NW, ept // 50, 50)
    acc1 = _sc_msg(n, e, h)(y1, src4, dst4)
    y2 = _tc_layer2(n, h)(acc1, y1, dinv, b_g1.reshape(1, h), W_g2)
    acc2 = _sc_msg(n, e, h)(y2, src4, dst4)
    hfeat, clu2d = _tc_kmeans(n, h)(acc2, y2, dinv, b_g2.reshape(1, h),
                                    batch.reshape(n, 1))
    clu = clu2d.reshape(n)
    cxs, cntf, eid3 = _sc_stats(n, h, CC, e)(hfeat, clu, src3, dst3)
    cntp = cntf.reshape(NW, CC)
    acnt = _sc_ahist(CC, e)(eid3.reshape(e))
    A2 = acnt.reshape(CC, CC)
    return _tc_final(h, out_dim)(
        A2, cxs, cntp, W_proj, b_proj.reshape(1, h), W_p1, b_p1.reshape(1, h),
        W_p2, b_p2.reshape(1, h), W_m1, b_m1.reshape(1, h),
        W_m2, b_m2.reshape(1, out_dim))
